# Initial kernel scaffold; baseline (speedup 1.0000x reference)
import jax
import jax.numpy as jnp
from jax.experimental import pallas as pl


def kernel(nodes, edges, init_node_w, init_node_b, init_edge_w, init_edge_b, l0_U_w, l0_U_b, l0_V_w, l0_V_b, l0_A_w, l0_A_b, l0_B_w, l0_B_b, l0_C_w, l0_C_b, l0_norm_h_gamma, l0_norm_h_beta, l0_norm_e_gamma, l0_norm_e_beta, l1_U_w, l1_U_b, l1_V_w, l1_V_b, l1_A_w, l1_A_b, l1_B_w, l1_B_b, l1_C_w, l1_C_b, l1_norm_h_gamma, l1_norm_h_beta, l1_norm_e_gamma, l1_norm_e_beta):
    raise NotImplementedError("write your pallas kernel here")


# speedup vs baseline: 2.3383x; 2.3383x over previous
import functools

import jax
import jax.numpy as jnp
from jax.experimental import pallas as pl
from jax.experimental.pallas import tpu as pltpu

_VMEM_LIMIT = 48 * 1024 * 1024
_H = 32


def _node_init_body(x_ref, wn_ref, bn_ref, wu_ref, bu_ref, h_ref, uvab_ref):
    h = (jnp.dot(x_ref[...], wn_ref[...], preferred_element_type=jnp.float32)
         + bn_ref[...])
    h_ref[...] = h
    uvab_ref[...] = (
        jnp.dot(h, wu_ref[...], preferred_element_type=jnp.float32)
        + bu_ref[...])


def _node_init(x2d, wn, bn, wu, bu):
    m, k = x2d.shape
    tm = m // 8 if m % 8 == 0 else m
    return pl.pallas_call(
        _node_init_body,
        grid=(m // tm,),
        out_shape=(jax.ShapeDtypeStruct((m, _H), jnp.float32),
                   jax.ShapeDtypeStruct((m, 4 * _H), jnp.float32)),
        in_specs=[
            pl.BlockSpec((tm, k), lambda i: (i, 0)),
            pl.BlockSpec((k, _H), lambda i: (0, 0)),
            pl.BlockSpec((1, _H), lambda i: (0, 0)),
            pl.BlockSpec((_H, 4 * _H), lambda i: (0, 0)),
            pl.BlockSpec((1, 4 * _H), lambda i: (0, 0)),
        ],
        out_specs=(pl.BlockSpec((tm, _H), lambda i: (i, 0)),
                   pl.BlockSpec((tm, 4 * _H), lambda i: (i, 0))),
        compiler_params=pltpu.CompilerParams(
            dimension_semantics=("parallel",), vmem_limit_bytes=_VMEM_LIMIT),
    )(x2d, wn, bn.reshape(1, _H), wu, bu.reshape(1, 4 * _H))


def _node_update_body(hp_ref, hr_ref, s_ref, t_ref, wu_ref, bu_ref,
                      h_ref, uvab_ref):
    y = hp_ref[...] * s_ref[...] + t_ref[...]
    h = hr_ref[...] + jnp.maximum(y, 0.0)
    h_ref[...] = h
    uvab_ref[...] = (
        jnp.dot(h, wu_ref[...], preferred_element_type=jnp.float32)
        + bu_ref[...])


def _node_update(hpre2d, hres2d, scale, shift, wu, bu):
    m = hpre2d.shape[0]
    tm = m // 8 if m % 8 == 0 else m
    row = pl.BlockSpec((tm, _H), lambda i: (i, 0))
    vec = pl.BlockSpec((1, _H), lambda i: (0, 0))
    return pl.pallas_call(
        _node_update_body,
        grid=(m // tm,),
        out_shape=(jax.ShapeDtypeStruct((m, _H), jnp.float32),
                   jax.ShapeDtypeStruct((m, 4 * _H), jnp.float32)),
        in_specs=[row, row, vec, vec,
                  pl.BlockSpec((_H, 4 * _H), lambda i: (0, 0)),
                  pl.BlockSpec((1, 4 * _H), lambda i: (0, 0))],
        out_specs=(row, pl.BlockSpec((tm, 4 * _H), lambda i: (i, 0))),
        compiler_params=pltpu.CompilerParams(
            dimension_semantics=("parallel",), vmem_limit_bytes=_VMEM_LIMIT),
    )(hpre2d, hres2d, scale.reshape(1, _H), shift.reshape(1, _H),
      wu, bu.reshape(1, 4 * _H))


def _fold_lanes_sum(row, P):
    acc = row[:, 0:_H]
    for p in range(1, P):
        acc = acc + row[:, p * _H:(p + 1) * _H]
    return acc


def _gate_aggregate(e_new, vhg, u, N, P):
    gated = jax.nn.sigmoid(e_new) * vhg[None, :, :]
    part = jnp.sum(gated, axis=1)
    agg = part[:, 0:_H]
    for p in range(1, P):
        agg = agg + part[:, p * _H:(p + 1) * _H]
    return u + agg * (1.0 / N)


def _write_stats(st_ref, h_pre, e_new2, P):
    es = jnp.sum(e_new2, axis=0, keepdims=True)
    eq = jnp.sum(e_new2 * e_new2, axis=0, keepdims=True)
    st_ref[0, 0] = jnp.concatenate(
        [jnp.sum(h_pre, axis=0, keepdims=True),
         jnp.sum(h_pre * h_pre, axis=0, keepdims=True),
         _fold_lanes_sum(es, P), _fold_lanes_sum(eq, P)], axis=0)


def _pass_a_body(epk_ref, u_ref, b_ref, ahg_ref, vhg_ref, wce_ref, bce_ref,
                 hpre_ref, st_ref, *, ti, G, P, N):
    C = P * _H
    ce = (jnp.dot(epk_ref[0], wce_ref[...],
                  preferred_element_type=jnp.float32) + bce_ref[...])
    bh_c = jnp.concatenate([b_ref[0]] * P, axis=1)
    e_new = ce.reshape(ti, G, C) + ahg_ref[0][None, :, :] + bh_c[:, None, :]
    h_pre = _gate_aggregate(e_new, vhg_ref[0], u_ref[0], N, P)
    hpre_ref[0] = h_pre
    _write_stats(st_ref, h_pre, e_new.reshape(ti * G, C), P)


def _pass_a(epk, u, b, ahg, vhg, wce, bce, *, ti, G, P, N):
    B = epk.shape[0]
    C = P * _H
    TR = ti * G
    n_it = N // ti
    body = functools.partial(_pass_a_body, ti=ti, G=G, P=P, N=N)
    return pl.pallas_call(
        body,
        grid=(B, n_it),
        out_shape=(jax.ShapeDtypeStruct((B, N, _H), jnp.float32),
                   jax.ShapeDtypeStruct((B, n_it, 4, _H), jnp.float32)),
        in_specs=[
            pl.BlockSpec((1, TR, 2 * P), lambda bb, it: (bb, it, 0)),
            pl.BlockSpec((1, ti, _H), lambda bb, it: (bb, it, 0)),
            pl.BlockSpec((1, ti, _H), lambda bb, it: (bb, it, 0)),
            pl.BlockSpec((1, G, C), lambda bb, it: (bb, 0, 0)),
            pl.BlockSpec((1, G, C), lambda bb, it: (bb, 0, 0)),
            pl.BlockSpec((2 * P, C), lambda bb, it: (0, 0)),
            pl.BlockSpec((1, C), lambda bb, it: (0, 0)),
        ],
        out_specs=(
            pl.BlockSpec((1, ti, _H), lambda bb, it: (bb, it, 0)),
            pl.BlockSpec((1, 1, 4, _H), lambda bb, it: (bb, it, 0, 0)),
        ),
        compiler_params=pltpu.CompilerParams(
            dimension_semantics=("parallel", "parallel"),
            vmem_limit_bytes=_VMEM_LIMIT),
    )(epk, u, b, ahg, vhg, wce, bce)


def _pass_b_body(epk_ref, b0_ref, ahg0_ref, w01_ref, b01_ref, se0_ref, te0_ref,
                 u1_ref, b1_ref, ahg1_ref, vhg1_ref, wc1_ref, bc1_ref,
                 hpre_ref, st_ref, *, ti, G, P, N):
    C = P * _H
    both = (jnp.dot(epk_ref[0], w01_ref[...],
                    preferred_element_type=jnp.float32) + b01_ref[...])
    e0 = both[:, 0:C]
    bh0 = jnp.concatenate([b0_ref[0]] * P, axis=1)
    e_new0 = (both[:, C:2 * C].reshape(ti, G, C)
              + ahg0_ref[0][None, :, :] + bh0[:, None, :]).reshape(ti * G, C)
    e1 = e0 + jnp.maximum(e_new0 * se0_ref[...] + te0_ref[...], 0.0)
    ce1 = (jnp.dot(e1, wc1_ref[...],
                   preferred_element_type=jnp.float32) + bc1_ref[...])
    bh1 = jnp.concatenate([b1_ref[0]] * P, axis=1)
    e_new1 = ce1.reshape(ti, G, C) + ahg1_ref[0][None, :, :] + bh1[:, None, :]
    h_pre = _gate_aggregate(e_new1, vhg1_ref[0], u1_ref[0], N, P)
    hpre_ref[0] = h_pre
    _write_stats(st_ref, h_pre, e_new1.reshape(ti * G, C), P)


def _pass_b(epk, b0, ahg0, w01, b01, se0, te0, u1, b1, ahg1, vhg1, wc1, bc1,
            *, ti, G, P, N):
    B = epk.shape[0]
    C = P * _H
    TR = ti * G
    n_it = N // ti
    body = functools.partial(_pass_b_body, ti=ti, G=G, P=P, N=N)
    vecC = pl.BlockSpec((1, C), lambda bb, it: (0, 0))
    rows = pl.BlockSpec((1, ti, _H), lambda bb, it: (bb, it, 0))
    batG = pl.BlockSpec((1, G, C), lambda bb, it: (bb, 0, 0))
    return pl.pallas_call(
        body,
        grid=(B, n_it),
        out_shape=(jax.ShapeDtypeStruct((B, N, _H), jnp.float32),
                   jax.ShapeDtypeStruct((B, n_it, 4, _H), jnp.float32)),
        in_specs=[
            pl.BlockSpec((1, TR, 2 * P), lambda bb, it: (bb, it, 0)),
            rows, batG,
            pl.BlockSpec((2 * P, 2 * C), lambda bb, it: (0, 0)),
            pl.BlockSpec((1, 2 * C), lambda bb, it: (0, 0)),
            vecC, vecC,
            rows, rows, batG, batG,
            pl.BlockSpec((C, C), lambda bb, it: (0, 0)),
            vecC,
        ],
        out_specs=(
            rows,
            pl.BlockSpec((1, 1, 4, _H), lambda bb, it: (bb, it, 0, 0)),
        ),
        compiler_params=pltpu.CompilerParams(
            dimension_semantics=("parallel", "parallel"),
            vmem_limit_bytes=_VMEM_LIMIT),
    )(epk, b0, ahg0, w01, b01, se0, te0, u1, b1, ahg1, vhg1, wc1, bc1)


def _pass_c_body(epk_ref, b0_ref, ahg0_ref, w01_ref, b01_ref, se0_ref, te0_ref,
                 b1_ref, ahg1_ref, wc1_ref, bc1_ref, se1_ref, te1_ref,
                 hp1_ref, hr_ref, sh1_ref, th1_ref,
                 eout_ref, hout_ref, *, ti, G, P):
    C = P * _H
    both = (jnp.dot(epk_ref[0], w01_ref[...],
                    preferred_element_type=jnp.float32) + b01_ref[...])
    e0 = both[:, 0:C]
    bh0 = jnp.concatenate([b0_ref[0]] * P, axis=1)
    e_new0 = (both[:, C:2 * C].reshape(ti, G, C)
              + ahg0_ref[0][None, :, :] + bh0[:, None, :]).reshape(ti * G, C)
    e1 = e0 + jnp.maximum(e_new0 * se0_ref[...] + te0_ref[...], 0.0)
    ce1 = (jnp.dot(e1, wc1_ref[...],
                   preferred_element_type=jnp.float32) + bc1_ref[...])
    bh1 = jnp.concatenate([b1_ref[0]] * P, axis=1)
    e_new1 = (ce1.reshape(ti, G, C)
              + ahg1_ref[0][None, :, :] + bh1[:, None, :]).reshape(ti * G, C)
    eout_ref[0] = e1 + jnp.maximum(e_new1 * se1_ref[...] + te1_ref[...], 0.0)
    yh = hp1_ref[0] * sh1_ref[...] + th1_ref[...]
    hout_ref[0] = hr_ref[0] + jnp.maximum(yh, 0.0)


def _pass_c(epk, b0, ahg0, w01, b01, se0, te0, b1, ahg1, wc1, bc1, se1, te1,
            hpre1, hres, sh1, th1, *, ti, G, P, N):
    B = epk.shape[0]
    C = P * _H
    TR = ti * G
    n_it = N // ti
    body = functools.partial(_pass_c_body, ti=ti, G=G, P=P)
    vecC = pl.BlockSpec((1, C), lambda bb, it: (0, 0))
    vecH = pl.BlockSpec((1, _H), lambda bb, it: (0, 0))
    rows = pl.BlockSpec((1, ti, _H), lambda bb, it: (bb, it, 0))
    batG = pl.BlockSpec((1, G, C), lambda bb, it: (bb, 0, 0))
    return pl.pallas_call(
        body,
        grid=(B, n_it),
        out_shape=(jax.ShapeDtypeStruct((B, N * G, C), jnp.float32),
                   jax.ShapeDtypeStruct((B, N, _H), jnp.float32)),
        in_specs=[
            pl.BlockSpec((1, TR, 2 * P), lambda bb, it: (bb, it, 0)),
            rows, batG,
            pl.BlockSpec((2 * P, 2 * C), lambda bb, it: (0, 0)),
            pl.BlockSpec((1, 2 * C), lambda bb, it: (0, 0)),
            vecC, vecC,
            rows, batG,
            pl.BlockSpec((C, C), lambda bb, it: (0, 0)),
            vecC, vecC, vecC,
            rows, rows, vecH, vecH,
        ],
        out_specs=(
            pl.BlockSpec((1, TR, C), lambda bb, it: (bb, it, 0)),
            rows,
        ),
        compiler_params=pltpu.CompilerParams(
            dimension_semantics=("parallel", "parallel"),
            vmem_limit_bytes=_VMEM_LIMIT),
    )(epk, b0, ahg0, w01, b01, se0, te0, b1, ahg1, wc1, bc1, se1, te1,
      hpre1, hres, sh1, th1)


def _bn_fold(stats, gamma_h, beta_h, gamma_e, beta_e, B, N, eps=1e-5):
    mh = float(B * N)
    h_mean = jnp.sum(stats[:, :, 0, :], axis=(0, 1)) / mh
    h_var = jnp.maximum(
        jnp.sum(stats[:, :, 1, :], axis=(0, 1)) / mh - h_mean * h_mean, 0.0)
    h_scale = gamma_h * jax.lax.rsqrt(h_var + eps)
    h_shift = beta_h - h_mean * h_scale
    me = float(B * N * N)
    e_mean = jnp.sum(stats[:, :, 2, :], axis=(0, 1)) / me
    e_var = jnp.maximum(
        jnp.sum(stats[:, :, 3, :], axis=(0, 1)) / me - e_mean * e_mean, 0.0)
    e_scale = gamma_e * jax.lax.rsqrt(e_var + eps)
    e_shift = beta_e - e_mean * e_scale
    return h_scale, h_shift, e_scale, e_shift


def kernel(nodes, edges,
           init_node_w, init_node_b, init_edge_w, init_edge_b,
           l0_U_w, l0_U_b, l0_V_w, l0_V_b, l0_A_w, l0_A_b,
           l0_B_w, l0_B_b, l0_C_w, l0_C_b,
           l0_norm_h_gamma, l0_norm_h_beta, l0_norm_e_gamma, l0_norm_e_beta,
           l1_U_w, l1_U_b, l1_V_w, l1_V_b, l1_A_w, l1_A_b,
           l1_B_w, l1_B_b, l1_C_w, l1_C_b,
           l1_norm_h_gamma, l1_norm_h_beta, l1_norm_e_gamma, l1_norm_e_beta):
    B, N, node_dim = nodes.shape
    H = _H
    P = 8
    G = N // P
    C = P * H
    ti = 32 if N % 32 == 0 else N
    f32 = jnp.float32

    eyeP = jnp.eye(P, dtype=f32)
    weP = jnp.kron(eyeP, init_edge_w)
    beP = jnp.tile(init_edge_b, P).reshape(1, C)
    wc0P = jnp.kron(eyeP, l0_C_w)
    bc0P = jnp.tile(l0_C_b, P).reshape(1, C)
    wc1P = jnp.kron(eyeP, l1_C_w)
    bc1P = jnp.tile(l1_C_b, P).reshape(1, C)
    wce0 = weP @ wc0P
    bce0 = beP @ wc0P + bc0P
    w01 = jnp.concatenate([weP, wce0], axis=1)
    b01 = jnp.concatenate([beP, bce0], axis=1)

    epk = edges.reshape(B, N * G, 2 * P)

    wu0 = jnp.concatenate([l0_U_w, l0_V_w, l0_A_w, l0_B_w], axis=1)
    bu0 = jnp.concatenate([l0_U_b, l0_V_b, l0_A_b, l0_B_b], axis=0)
    wu1 = jnp.concatenate([l1_U_w, l1_V_w, l1_A_w, l1_B_w], axis=1)
    bu1 = jnp.concatenate([l1_U_b, l1_V_b, l1_A_b, l1_B_b], axis=0)

    h0_2d, uvab0 = _node_init(nodes.reshape(B * N, node_dim),
                              init_node_w, init_node_b, wu0, bu0)
    u0 = uvab0[:, 0:H].reshape(B, N, H)
    vhg0 = uvab0[:, H:2 * H].reshape(B, G, C)
    ahg0 = uvab0[:, 2 * H:3 * H].reshape(B, G, C)
    b0 = uvab0[:, 3 * H:4 * H].reshape(B, N, H)

    hpre0, st0 = _pass_a(epk, u0, b0, ahg0, vhg0, wce0, bce0,
                         ti=ti, G=G, P=P, N=N)
    hs0, ht0, es0, et0 = _bn_fold(st0, l0_norm_h_gamma, l0_norm_h_beta,
                                  l0_norm_e_gamma, l0_norm_e_beta, B, N)
    se0 = jnp.tile(es0, P).reshape(1, C)
    te0 = jnp.tile(et0, P).reshape(1, C)

    h1_2d, uvab1 = _node_update(hpre0.reshape(B * N, H), h0_2d,
                                hs0, ht0, wu1, bu1)
    u1 = uvab1[:, 0:H].reshape(B, N, H)
    vhg1 = uvab1[:, H:2 * H].reshape(B, G, C)
    ahg1 = uvab1[:, 2 * H:3 * H].reshape(B, G, C)
    b1 = uvab1[:, 3 * H:4 * H].reshape(B, N, H)

    hpre1, st1 = _pass_b(epk, b0, ahg0, w01, b01, se0, te0,
                         u1, b1, ahg1, vhg1, wc1P, bc1P,
                         ti=ti, G=G, P=P, N=N)
    hs1, ht1, es1, et1 = _bn_fold(st1, l1_norm_h_gamma, l1_norm_h_beta,
                                  l1_norm_e_gamma, l1_norm_e_beta, B, N)
    se1 = jnp.tile(es1, P).reshape(1, C)
    te1 = jnp.tile(et1, P).reshape(1, C)

    e_out, h_out = _pass_c(epk, b0, ahg0, w01, b01, se0, te0,
                           b1, ahg1, wc1P, bc1P, se1, te1,
                           hpre1, h1_2d.reshape(B, N, H),
                           hs1.reshape(1, H), ht1.reshape(1, H),
                           ti=ti, G=G, P=P, N=N)
    return h_out, e_out.reshape(B, N, N, H)


# speedup vs baseline: 2.4369x; 1.0422x over previous
import functools

import jax
import jax.numpy as jnp
from jax.experimental import pallas as pl
from jax.experimental.pallas import tpu as pltpu

_VMEM_LIMIT = 48 * 1024 * 1024
_H = 32


def _node_init_body(x_ref, wn_ref, bn_ref, wu_ref, bu_ref, h_ref, uvab_ref):
    h = (jnp.dot(x_ref[...], wn_ref[...], preferred_element_type=jnp.float32)
         + bn_ref[...])
    h_ref[...] = h
    uvab_ref[...] = (
        jnp.dot(h, wu_ref[...], preferred_element_type=jnp.float32)
        + bu_ref[...])


def _node_init(x2d, wn, bn, wu, bu):
    m, k = x2d.shape
    tm = m // 8 if m % 8 == 0 else m
    return pl.pallas_call(
        _node_init_body,
        grid=(m // tm,),
        out_shape=(jax.ShapeDtypeStruct((m, _H), jnp.float32),
                   jax.ShapeDtypeStruct((m, 4 * _H), jnp.float32)),
        in_specs=[
            pl.BlockSpec((tm, k), lambda i: (i, 0)),
            pl.BlockSpec((k, _H), lambda i: (0, 0)),
            pl.BlockSpec((1, _H), lambda i: (0, 0)),
            pl.BlockSpec((_H, 4 * _H), lambda i: (0, 0)),
            pl.BlockSpec((1, 4 * _H), lambda i: (0, 0)),
        ],
        out_specs=(pl.BlockSpec((tm, _H), lambda i: (i, 0)),
                   pl.BlockSpec((tm, 4 * _H), lambda i: (i, 0))),
        compiler_params=pltpu.CompilerParams(
            dimension_semantics=("parallel",), vmem_limit_bytes=_VMEM_LIMIT),
    )(x2d, wn, bn.reshape(1, _H), wu, bu.reshape(1, 4 * _H))


def _node_update_body(hp_ref, hr_ref, s_ref, t_ref, wu_ref, bu_ref,
                      h_ref, uvab_ref):
    y = hp_ref[...] * s_ref[...] + t_ref[...]
    h = hr_ref[...] + jnp.maximum(y, 0.0)
    h_ref[...] = h
    uvab_ref[...] = (
        jnp.dot(h, wu_ref[...], preferred_element_type=jnp.float32)
        + bu_ref[...])


def _node_update(hpre2d, hres2d, scale, shift, wu, bu):
    m = hpre2d.shape[0]
    tm = m // 8 if m % 8 == 0 else m
    row = pl.BlockSpec((tm, _H), lambda i: (i, 0))
    vec = pl.BlockSpec((1, _H), lambda i: (0, 0))
    return pl.pallas_call(
        _node_update_body,
        grid=(m // tm,),
        out_shape=(jax.ShapeDtypeStruct((m, _H), jnp.float32),
                   jax.ShapeDtypeStruct((m, 4 * _H), jnp.float32)),
        in_specs=[row, row, vec, vec,
                  pl.BlockSpec((_H, 4 * _H), lambda i: (0, 0)),
                  pl.BlockSpec((1, 4 * _H), lambda i: (0, 0))],
        out_specs=(row, pl.BlockSpec((tm, 4 * _H), lambda i: (i, 0))),
        compiler_params=pltpu.CompilerParams(
            dimension_semantics=("parallel",), vmem_limit_bytes=_VMEM_LIMIT),
    )(hpre2d, hres2d, scale.reshape(1, _H), shift.reshape(1, _H),
      wu, bu.reshape(1, 4 * _H))


def _fold_lanes_sum(row, P):
    acc = row[:, 0:_H]
    for p in range(1, P):
        acc = acc + row[:, p * _H:(p + 1) * _H]
    return acc


def _gate_aggregate(e_new, vhg, u, N, P):
    gated = jax.nn.sigmoid(e_new) * vhg[None, :, :]
    part = jnp.sum(gated, axis=1)
    agg = part[:, 0:_H]
    for p in range(1, P):
        agg = agg + part[:, p * _H:(p + 1) * _H]
    return u + agg * (1.0 / N)


def _write_stats(st_ref, h_pre, e_new2, P):
    es = jnp.sum(e_new2, axis=0, keepdims=True)
    eq = jnp.sum(e_new2 * e_new2, axis=0, keepdims=True)
    st_ref[0, 0] = jnp.concatenate(
        [jnp.sum(h_pre, axis=0, keepdims=True),
         jnp.sum(h_pre * h_pre, axis=0, keepdims=True),
         _fold_lanes_sum(es, P), _fold_lanes_sum(eq, P)], axis=0)


def _pass_a_body(epk_ref, u_ref, b_ref, ahg_ref, vhg_ref, wce_ref, bce_ref,
                 hpre_ref, st_ref, *, ti, G, P, N):
    C = P * _H
    ce = (jnp.dot(epk_ref[0], wce_ref[...],
                  preferred_element_type=jnp.float32) + bce_ref[...])
    bh_c = jnp.concatenate([b_ref[0]] * P, axis=1)
    e_new = ce.reshape(ti, G, C) + ahg_ref[0][None, :, :] + bh_c[:, None, :]
    h_pre = _gate_aggregate(e_new, vhg_ref[0], u_ref[0], N, P)
    hpre_ref[0] = h_pre
    _write_stats(st_ref, h_pre, e_new.reshape(ti * G, C), P)


def _pass_a(epk, u, b, ahg, vhg, wce, bce, *, ti, G, P, N):
    B = epk.shape[0]
    C = P * _H
    TR = ti * G
    n_it = N // ti
    body = functools.partial(_pass_a_body, ti=ti, G=G, P=P, N=N)
    return pl.pallas_call(
        body,
        grid=(B, n_it),
        out_shape=(jax.ShapeDtypeStruct((B, N, _H), jnp.float32),
                   jax.ShapeDtypeStruct((B, n_it, 4, _H), jnp.float32)),
        in_specs=[
            pl.BlockSpec((1, ti * G, 2 * P), lambda bb, it: (bb, it, 0)),
            pl.BlockSpec((1, ti, _H), lambda bb, it: (bb, it, 0)),
            pl.BlockSpec((1, ti, _H), lambda bb, it: (bb, it, 0)),
            pl.BlockSpec((1, G, C), lambda bb, it: (bb, 0, 0)),
            pl.BlockSpec((1, G, C), lambda bb, it: (bb, 0, 0)),
            pl.BlockSpec((2 * P, C), lambda bb, it: (0, 0)),
            pl.BlockSpec((1, C), lambda bb, it: (0, 0)),
        ],
        out_specs=(
            pl.BlockSpec((1, ti, _H), lambda bb, it: (bb, it, 0)),
            pl.BlockSpec((1, 1, 4, _H), lambda bb, it: (bb, it, 0, 0)),
        ),
        compiler_params=pltpu.CompilerParams(
            dimension_semantics=("parallel", "parallel"),
            vmem_limit_bytes=_VMEM_LIMIT),
    )(epk, u, b, ahg, vhg, wce, bce)


def _pass_b_body(epk_ref, b0_ref, ahg0_ref, w01_ref, b01_ref, se0_ref, te0_ref,
                 u1_ref, b1_ref, ahg1_ref, vhg1_ref, wc1_ref, bc1_ref,
                 hpre_ref, st_ref, *, ti, G, P, N):
    C = P * _H
    both = (jnp.dot(epk_ref[0], w01_ref[...],
                    preferred_element_type=jnp.float32) + b01_ref[...])
    e0 = both[:, 0:C]
    bh0 = jnp.concatenate([b0_ref[0]] * P, axis=1)
    e_new0 = (both[:, C:2 * C].reshape(ti, G, C)
              + ahg0_ref[0][None, :, :] + bh0[:, None, :]).reshape(ti * G, C)
    e1 = e0 + jnp.maximum(e_new0 * se0_ref[...] + te0_ref[...], 0.0)
    ce1 = (jnp.dot(e1, wc1_ref[...],
                   preferred_element_type=jnp.float32) + bc1_ref[...])
    bh1 = jnp.concatenate([b1_ref[0]] * P, axis=1)
    e_new1 = ce1.reshape(ti, G, C) + ahg1_ref[0][None, :, :] + bh1[:, None, :]
    h_pre = _gate_aggregate(e_new1, vhg1_ref[0], u1_ref[0], N, P)
    hpre_ref[0] = h_pre
    _write_stats(st_ref, h_pre, e_new1.reshape(ti * G, C), P)


def _pass_b(epk, b0, ahg0, w01, b01, se0, te0, u1, b1, ahg1, vhg1, wc1, bc1,
            *, ti, G, P, N):
    B = epk.shape[0]
    C = P * _H
    TR = ti * G
    n_it = N // ti
    body = functools.partial(_pass_b_body, ti=ti, G=G, P=P, N=N)
    vecC = pl.BlockSpec((1, C), lambda bb, it: (0, 0))
    rows = pl.BlockSpec((1, ti, _H), lambda bb, it: (bb, it, 0))
    batG = pl.BlockSpec((1, G, C), lambda bb, it: (bb, 0, 0))
    return pl.pallas_call(
        body,
        grid=(B, n_it),
        out_shape=(jax.ShapeDtypeStruct((B, N, _H), jnp.float32),
                   jax.ShapeDtypeStruct((B, n_it, 4, _H), jnp.float32)),
        in_specs=[
            pl.BlockSpec((1, ti * G, 2 * P), lambda bb, it: (bb, it, 0)),
            rows, batG,
            pl.BlockSpec((2 * P, 2 * C), lambda bb, it: (0, 0)),
            pl.BlockSpec((1, 2 * C), lambda bb, it: (0, 0)),
            vecC, vecC,
            rows, rows, batG, batG,
            pl.BlockSpec((C, C), lambda bb, it: (0, 0)),
            vecC,
        ],
        out_specs=(
            rows,
            pl.BlockSpec((1, 1, 4, _H), lambda bb, it: (bb, it, 0, 0)),
        ),
        compiler_params=pltpu.CompilerParams(
            dimension_semantics=("parallel", "parallel"),
            vmem_limit_bytes=_VMEM_LIMIT),
    )(epk, b0, ahg0, w01, b01, se0, te0, u1, b1, ahg1, vhg1, wc1, bc1)


def _pass_c_body(epk_ref, b0_ref, ahg0_ref, w01_ref, b01_ref, se0_ref, te0_ref,
                 b1_ref, ahg1_ref, wc1_ref, bc1_ref, se1_ref, te1_ref,
                 hp1_ref, hr_ref, sh1_ref, th1_ref,
                 eout_ref, hout_ref, *, ti, G, P, N):
    C = P * _H
    both = (jnp.dot(epk_ref[0], w01_ref[...],
                    preferred_element_type=jnp.float32) + b01_ref[...])
    e0 = both[:, 0:C]
    bh0 = jnp.concatenate([b0_ref[0]] * P, axis=1)
    e_new0 = (both[:, C:2 * C].reshape(ti, G, C)
              + ahg0_ref[0][None, :, :] + bh0[:, None, :]).reshape(ti * G, C)
    e1 = e0 + jnp.maximum(e_new0 * se0_ref[...] + te0_ref[...], 0.0)
    ce1 = (jnp.dot(e1, wc1_ref[...],
                   preferred_element_type=jnp.float32) + bc1_ref[...])
    bh1 = jnp.concatenate([b1_ref[0]] * P, axis=1)
    e_new1 = (ce1.reshape(ti, G, C)
              + ahg1_ref[0][None, :, :] + bh1[:, None, :]).reshape(ti * G, C)
    eo = (e1 + jnp.maximum(e_new1 * se1_ref[...] + te1_ref[...], 0.0)
          ).reshape(ti, G, C)
    for p in range(P):
        eout_ref[0, :, p * G:(p + 1) * G, :] = eo[:, :, p * _H:(p + 1) * _H]
    yh = hp1_ref[0] * sh1_ref[...] + th1_ref[...]
    hout_ref[0] = hr_ref[0] + jnp.maximum(yh, 0.0)


def _pass_c(epk, b0, ahg0, w01, b01, se0, te0, b1, ahg1, wc1, bc1, se1, te1,
            hpre1, hres, sh1, th1, *, ti, G, P, N):
    B = epk.shape[0]
    C = P * _H
    n_it = N // ti
    body = functools.partial(_pass_c_body, ti=ti, G=G, P=P, N=N)
    vecC = pl.BlockSpec((1, C), lambda bb, it: (0, 0))
    vecH = pl.BlockSpec((1, _H), lambda bb, it: (0, 0))
    rows = pl.BlockSpec((1, ti, _H), lambda bb, it: (bb, it, 0))
    batG = pl.BlockSpec((1, G, C), lambda bb, it: (bb, 0, 0))
    return pl.pallas_call(
        body,
        grid=(B, n_it),
        out_shape=(jax.ShapeDtypeStruct((B, N, N, _H), jnp.float32),
                   jax.ShapeDtypeStruct((B, N, _H), jnp.float32)),
        in_specs=[
            pl.BlockSpec((1, ti * G, 2 * P), lambda bb, it: (bb, it, 0)),
            rows, batG,
            pl.BlockSpec((2 * P, 2 * C), lambda bb, it: (0, 0)),
            pl.BlockSpec((1, 2 * C), lambda bb, it: (0, 0)),
            vecC, vecC,
            rows, batG,
            pl.BlockSpec((C, C), lambda bb, it: (0, 0)),
            vecC, vecC, vecC,
            rows, rows, vecH, vecH,
        ],
        out_specs=(
            pl.BlockSpec((1, ti, N, _H), lambda bb, it: (bb, it, 0, 0)),
            rows,
        ),
        compiler_params=pltpu.CompilerParams(
            dimension_semantics=("parallel", "parallel"),
            vmem_limit_bytes=_VMEM_LIMIT),
    )(epk, b0, ahg0, w01, b01, se0, te0, b1, ahg1, wc1, bc1, se1, te1,
      hpre1, hres, sh1, th1)


def _bn_fold(stats, gamma_h, beta_h, gamma_e, beta_e, B, N, eps=1e-5):
    mh = float(B * N)
    h_mean = jnp.sum(stats[:, :, 0, :], axis=(0, 1)) / mh
    h_var = jnp.maximum(
        jnp.sum(stats[:, :, 1, :], axis=(0, 1)) / mh - h_mean * h_mean, 0.0)
    h_scale = gamma_h * jax.lax.rsqrt(h_var + eps)
    h_shift = beta_h - h_mean * h_scale
    me = float(B * N * N)
    e_mean = jnp.sum(stats[:, :, 2, :], axis=(0, 1)) / me
    e_var = jnp.maximum(
        jnp.sum(stats[:, :, 3, :], axis=(0, 1)) / me - e_mean * e_mean, 0.0)
    e_scale = gamma_e * jax.lax.rsqrt(e_var + eps)
    e_shift = beta_e - e_mean * e_scale
    return h_scale, h_shift, e_scale, e_shift


def kernel(nodes, edges,
           init_node_w, init_node_b, init_edge_w, init_edge_b,
           l0_U_w, l0_U_b, l0_V_w, l0_V_b, l0_A_w, l0_A_b,
           l0_B_w, l0_B_b, l0_C_w, l0_C_b,
           l0_norm_h_gamma, l0_norm_h_beta, l0_norm_e_gamma, l0_norm_e_beta,
           l1_U_w, l1_U_b, l1_V_w, l1_V_b, l1_A_w, l1_A_b,
           l1_B_w, l1_B_b, l1_C_w, l1_C_b,
           l1_norm_h_gamma, l1_norm_h_beta, l1_norm_e_gamma, l1_norm_e_beta):
    B, N, node_dim = nodes.shape
    H = _H
    P = 8
    G = N // P
    C = P * H
    ti = 32 if N % 32 == 0 else N
    f32 = jnp.float32

    eyeP = jnp.eye(P, dtype=f32)
    weP = jnp.kron(eyeP, init_edge_w)
    beP = jnp.tile(init_edge_b, P).reshape(1, C)
    wc0P = jnp.kron(eyeP, l0_C_w)
    bc0P = jnp.tile(l0_C_b, P).reshape(1, C)
    wc1P = jnp.kron(eyeP, l1_C_w)
    bc1P = jnp.tile(l1_C_b, P).reshape(1, C)
    wce0 = weP @ wc0P
    bce0 = beP @ wc0P + bc0P
    w01 = jnp.concatenate([weP, wce0], axis=1)
    b01 = jnp.concatenate([beP, bce0], axis=1)

    epk = edges.reshape(B, N, P, G, 2).transpose(0, 1, 3, 2, 4).reshape(
        B, N * G, 2 * P)

    wu0 = jnp.concatenate([l0_U_w, l0_V_w, l0_A_w, l0_B_w], axis=1)
    bu0 = jnp.concatenate([l0_U_b, l0_V_b, l0_A_b, l0_B_b], axis=0)
    wu1 = jnp.concatenate([l1_U_w, l1_V_w, l1_A_w, l1_B_w], axis=1)
    bu1 = jnp.concatenate([l1_U_b, l1_V_b, l1_A_b, l1_B_b], axis=0)

    def pack_g(col):
        return col.reshape(B, P, G, H).transpose(0, 2, 1, 3).reshape(B, G, C)

    h0_2d, uvab0 = _node_init(nodes.reshape(B * N, node_dim),
                              init_node_w, init_node_b, wu0, bu0)
    u0 = uvab0[:, 0:H].reshape(B, N, H)
    vhg0 = pack_g(uvab0[:, H:2 * H])
    ahg0 = pack_g(uvab0[:, 2 * H:3 * H])
    b0 = uvab0[:, 3 * H:4 * H].reshape(B, N, H)

    hpre0, st0 = _pass_a(epk, u0, b0, ahg0, vhg0, wce0, bce0,
                         ti=ti, G=G, P=P, N=N)
    hs0, ht0, es0, et0 = _bn_fold(st0, l0_norm_h_gamma, l0_norm_h_beta,
                                  l0_norm_e_gamma, l0_norm_e_beta, B, N)
    se0 = jnp.tile(es0, P).reshape(1, C)
    te0 = jnp.tile(et0, P).reshape(1, C)

    h1_2d, uvab1 = _node_update(hpre0.reshape(B * N, H), h0_2d,
                                hs0, ht0, wu1, bu1)
    u1 = uvab1[:, 0:H].reshape(B, N, H)
    vhg1 = pack_g(uvab1[:, H:2 * H])
    ahg1 = pack_g(uvab1[:, 2 * H:3 * H])
    b1 = uvab1[:, 3 * H:4 * H].reshape(B, N, H)

    hpre1, st1 = _pass_b(epk, b0, ahg0, w01, b01, se0, te0,
                         u1, b1, ahg1, vhg1, wc1P, bc1P,
                         ti=ti, G=G, P=P, N=N)
    hs1, ht1, es1, et1 = _bn_fold(st1, l1_norm_h_gamma, l1_norm_h_beta,
                                  l1_norm_e_gamma, l1_norm_e_beta, B, N)
    se1 = jnp.tile(es1, P).reshape(1, C)
    te1 = jnp.tile(et1, P).reshape(1, C)

    e_out, h_out = _pass_c(epk, b0, ahg0, w01, b01, se0, te0,
                           b1, ahg1, wc1P, bc1P, se1, te1,
                           hpre1, h1_2d.reshape(B, N, H),
                           hs1.reshape(1, H), ht1.reshape(1, H),
                           ti=ti, G=G, P=P, N=N)
    return h_out, e_out


# speedup vs baseline: 2.4611x; 1.0099x over previous
import functools

import jax
import jax.numpy as jnp
from jax.experimental import pallas as pl
from jax.experimental.pallas import tpu as pltpu

_VMEM_LIMIT = 48 * 1024 * 1024
_H = 32


def _node_init_body(x_ref, wn_ref, bn_ref, wu_ref, bu_ref, h_ref, uvab_ref):
    h = (jnp.dot(x_ref[...], wn_ref[...], preferred_element_type=jnp.float32)
         + bn_ref[...])
    h_ref[...] = h
    uvab_ref[...] = (
        jnp.dot(h, wu_ref[...], preferred_element_type=jnp.float32)
        + bu_ref[...])


def _node_init(x2d, wn, bn, wu, bu):
    m, k = x2d.shape
    tm = m // 8 if m % 8 == 0 else m
    return pl.pallas_call(
        _node_init_body,
        grid=(m // tm,),
        out_shape=(jax.ShapeDtypeStruct((m, _H), jnp.float32),
                   jax.ShapeDtypeStruct((m, 4 * _H), jnp.float32)),
        in_specs=[
            pl.BlockSpec((tm, k), lambda i: (i, 0)),
            pl.BlockSpec((k, _H), lambda i: (0, 0)),
            pl.BlockSpec((1, _H), lambda i: (0, 0)),
            pl.BlockSpec((_H, 4 * _H), lambda i: (0, 0)),
            pl.BlockSpec((1, 4 * _H), lambda i: (0, 0)),
        ],
        out_specs=(pl.BlockSpec((tm, _H), lambda i: (i, 0)),
                   pl.BlockSpec((tm, 4 * _H), lambda i: (i, 0))),
        compiler_params=pltpu.CompilerParams(
            dimension_semantics=("parallel",), vmem_limit_bytes=_VMEM_LIMIT),
    )(x2d, wn, bn.reshape(1, _H), wu, bu.reshape(1, 4 * _H))


def _node_update_body(hp_ref, hr_ref, s_ref, t_ref, wu_ref, bu_ref,
                      h_ref, uvab_ref):
    y = hp_ref[...] * s_ref[...] + t_ref[...]
    h = hr_ref[...] + jnp.maximum(y, 0.0)
    h_ref[...] = h
    uvab_ref[...] = (
        jnp.dot(h, wu_ref[...], preferred_element_type=jnp.float32)
        + bu_ref[...])


def _node_update(hpre2d, hres2d, scale, shift, wu, bu):
    m = hpre2d.shape[0]
    tm = m // 8 if m % 8 == 0 else m
    row = pl.BlockSpec((tm, _H), lambda i: (i, 0))
    vec = pl.BlockSpec((1, _H), lambda i: (0, 0))
    return pl.pallas_call(
        _node_update_body,
        grid=(m // tm,),
        out_shape=(jax.ShapeDtypeStruct((m, _H), jnp.float32),
                   jax.ShapeDtypeStruct((m, 4 * _H), jnp.float32)),
        in_specs=[row, row, vec, vec,
                  pl.BlockSpec((_H, 4 * _H), lambda i: (0, 0)),
                  pl.BlockSpec((1, 4 * _H), lambda i: (0, 0))],
        out_specs=(row, pl.BlockSpec((tm, 4 * _H), lambda i: (i, 0))),
        compiler_params=pltpu.CompilerParams(
            dimension_semantics=("parallel",), vmem_limit_bytes=_VMEM_LIMIT),
    )(hpre2d, hres2d, scale.reshape(1, _H), shift.reshape(1, _H),
      wu, bu.reshape(1, 4 * _H))


def _fold_lanes_sum(row, P):
    acc = row[:, 0:_H]
    for p in range(1, P):
        acc = acc + row[:, p * _H:(p + 1) * _H]
    return acc


def _gate_aggregate(e_new, vhg, u, N, P):
    gated = jax.nn.sigmoid(e_new) * vhg[None, :, :]
    part = jnp.sum(gated, axis=1)
    agg = part[:, 0:_H]
    for p in range(1, P):
        agg = agg + part[:, p * _H:(p + 1) * _H]
    return u + agg * (1.0 / N)


def _write_stats(st_ref, h_pre, e_new2, P):
    es = jnp.sum(e_new2, axis=0, keepdims=True)
    eq = jnp.sum(e_new2 * e_new2, axis=0, keepdims=True)
    st_ref[0, 0] = jnp.concatenate(
        [jnp.sum(h_pre, axis=0, keepdims=True),
         jnp.sum(h_pre * h_pre, axis=0, keepdims=True),
         _fold_lanes_sum(es, P), _fold_lanes_sum(eq, P)], axis=0)


def _pass_a_body(epk_ref, u_ref, b_ref, ahg_ref, vhg_ref, wce_ref, bce_ref,
                 hpre_ref, st_ref, *, ti, G, P, N):
    C = P * _H
    ce = (jnp.dot(epk_ref[0], wce_ref[...],
                  preferred_element_type=jnp.float32) + bce_ref[...])
    bh_c = jnp.concatenate([b_ref[0]] * P, axis=1)
    e_new = ce.reshape(ti, G, C) + ahg_ref[0][None, :, :] + bh_c[:, None, :]
    h_pre = _gate_aggregate(e_new, vhg_ref[0], u_ref[0], N, P)
    hpre_ref[0] = h_pre
    _write_stats(st_ref, h_pre, e_new.reshape(ti * G, C), P)


def _pass_a(epk, u, b, ahg, vhg, wce, bce, *, ti, G, P, N):
    B = epk.shape[0]
    C = P * _H
    TR = ti * G
    n_it = N // ti
    body = functools.partial(_pass_a_body, ti=ti, G=G, P=P, N=N)
    return pl.pallas_call(
        body,
        grid=(B, n_it),
        out_shape=(jax.ShapeDtypeStruct((B, N, _H), jnp.float32),
                   jax.ShapeDtypeStruct((B, n_it, 4, _H), jnp.float32)),
        in_specs=[
            pl.BlockSpec((1, ti * G, 2 * P), lambda bb, it: (bb, it, 0)),
            pl.BlockSpec((1, ti, _H), lambda bb, it: (bb, it, 0)),
            pl.BlockSpec((1, ti, _H), lambda bb, it: (bb, it, 0)),
            pl.BlockSpec((1, G, C), lambda bb, it: (bb, 0, 0)),
            pl.BlockSpec((1, G, C), lambda bb, it: (bb, 0, 0)),
            pl.BlockSpec((2 * P, C), lambda bb, it: (0, 0)),
            pl.BlockSpec((1, C), lambda bb, it: (0, 0)),
        ],
        out_specs=(
            pl.BlockSpec((1, ti, _H), lambda bb, it: (bb, it, 0)),
            pl.BlockSpec((1, 1, 4, _H), lambda bb, it: (bb, it, 0, 0)),
        ),
        compiler_params=pltpu.CompilerParams(
            dimension_semantics=("parallel", "parallel"),
            vmem_limit_bytes=_VMEM_LIMIT),
    )(epk, u, b, ahg, vhg, wce, bce)


def _pass_b_body(epk_ref, b0_ref, ahg0_ref, w01_ref, b01_ref, se0_ref, te0_ref,
                 u1_ref, b1_ref, ahg1_ref, vhg1_ref, wc1_ref, bc1_ref,
                 hpre_ref, st_ref, *, ti, G, P, N):
    C = P * _H
    both = (jnp.dot(epk_ref[0], w01_ref[...],
                    preferred_element_type=jnp.float32) + b01_ref[...])
    e0 = both[:, 0:C]
    bh0 = jnp.concatenate([b0_ref[0]] * P, axis=1)
    e_new0 = (both[:, C:2 * C].reshape(ti, G, C)
              + ahg0_ref[0][None, :, :] + bh0[:, None, :]).reshape(ti * G, C)
    e1 = e0 + jnp.maximum(e_new0 * se0_ref[...] + te0_ref[...], 0.0)
    ce1 = (jnp.dot(e1, wc1_ref[...],
                   preferred_element_type=jnp.float32) + bc1_ref[...])
    bh1 = jnp.concatenate([b1_ref[0]] * P, axis=1)
    e_new1 = ce1.reshape(ti, G, C) + ahg1_ref[0][None, :, :] + bh1[:, None, :]
    h_pre = _gate_aggregate(e_new1, vhg1_ref[0], u1_ref[0], N, P)
    hpre_ref[0] = h_pre
    _write_stats(st_ref, h_pre, e_new1.reshape(ti * G, C), P)


def _pass_b(epk, b0, ahg0, w01, b01, se0, te0, u1, b1, ahg1, vhg1, wc1, bc1,
            *, ti, G, P, N):
    B = epk.shape[0]
    C = P * _H
    TR = ti * G
    n_it = N // ti
    body = functools.partial(_pass_b_body, ti=ti, G=G, P=P, N=N)
    vecC = pl.BlockSpec((1, C), lambda bb, it: (0, 0))
    rows = pl.BlockSpec((1, ti, _H), lambda bb, it: (bb, it, 0))
    batG = pl.BlockSpec((1, G, C), lambda bb, it: (bb, 0, 0))
    return pl.pallas_call(
        body,
        grid=(B, n_it),
        out_shape=(jax.ShapeDtypeStruct((B, N, _H), jnp.float32),
                   jax.ShapeDtypeStruct((B, n_it, 4, _H), jnp.float32)),
        in_specs=[
            pl.BlockSpec((1, ti * G, 2 * P), lambda bb, it: (bb, it, 0)),
            rows, batG,
            pl.BlockSpec((2 * P, 2 * C), lambda bb, it: (0, 0)),
            pl.BlockSpec((1, 2 * C), lambda bb, it: (0, 0)),
            vecC, vecC,
            rows, rows, batG, batG,
            pl.BlockSpec((C, C), lambda bb, it: (0, 0)),
            vecC,
        ],
        out_specs=(
            rows,
            pl.BlockSpec((1, 1, 4, _H), lambda bb, it: (bb, it, 0, 0)),
        ),
        compiler_params=pltpu.CompilerParams(
            dimension_semantics=("parallel", "parallel"),
            vmem_limit_bytes=_VMEM_LIMIT),
    )(epk, b0, ahg0, w01, b01, se0, te0, u1, b1, ahg1, vhg1, wc1, bc1)


def _pass_c_body(epk_ref, b0_ref, ahg0_ref, w01_ref, b01_ref, se0_ref, te0_ref,
                 b1_ref, ahg1_ref, wc1_ref, bc1_ref, se1_ref, te1_ref,
                 hp1_ref, hr_ref, sh1_ref, th1_ref,
                 eout_ref, hout_ref, *, ti, G, P, N):
    C = P * _H
    both = (jnp.dot(epk_ref[0], w01_ref[...],
                    preferred_element_type=jnp.float32) + b01_ref[...])
    e0 = both[:, 0:C]
    bh0 = jnp.concatenate([b0_ref[0]] * P, axis=1)
    e_new0 = (both[:, C:2 * C].reshape(ti, G, C)
              + ahg0_ref[0][None, :, :] + bh0[:, None, :]).reshape(ti * G, C)
    e1 = e0 + jnp.maximum(e_new0 * se0_ref[...] + te0_ref[...], 0.0)
    ce1 = (jnp.dot(e1, wc1_ref[...],
                   preferred_element_type=jnp.float32) + bc1_ref[...])
    bh1 = jnp.concatenate([b1_ref[0]] * P, axis=1)
    e_new1 = (ce1.reshape(ti, G, C)
              + ahg1_ref[0][None, :, :] + bh1[:, None, :]).reshape(ti * G, C)
    eo = (e1 + jnp.maximum(e_new1 * se1_ref[...] + te1_ref[...], 0.0)
          ).reshape(ti, G, C)
    for p in range(P):
        eout_ref[0, :, p::P, :] = eo[:, :, p * _H:(p + 1) * _H]
    yh = hp1_ref[0] * sh1_ref[...] + th1_ref[...]
    hout_ref[0] = hr_ref[0] + jnp.maximum(yh, 0.0)


def _pass_c(epk, b0, ahg0, w01, b01, se0, te0, b1, ahg1, wc1, bc1, se1, te1,
            hpre1, hres, sh1, th1, *, ti, G, P, N):
    B = epk.shape[0]
    C = P * _H
    n_it = N // ti
    body = functools.partial(_pass_c_body, ti=ti, G=G, P=P, N=N)
    vecC = pl.BlockSpec((1, C), lambda bb, it: (0, 0))
    vecH = pl.BlockSpec((1, _H), lambda bb, it: (0, 0))
    rows = pl.BlockSpec((1, ti, _H), lambda bb, it: (bb, it, 0))
    batG = pl.BlockSpec((1, G, C), lambda bb, it: (bb, 0, 0))
    return pl.pallas_call(
        body,
        grid=(B, n_it),
        out_shape=(jax.ShapeDtypeStruct((B, N, N, _H), jnp.float32),
                   jax.ShapeDtypeStruct((B, N, _H), jnp.float32)),
        in_specs=[
            pl.BlockSpec((1, ti * G, 2 * P), lambda bb, it: (bb, it, 0)),
            rows, batG,
            pl.BlockSpec((2 * P, 2 * C), lambda bb, it: (0, 0)),
            pl.BlockSpec((1, 2 * C), lambda bb, it: (0, 0)),
            vecC, vecC,
            rows, batG,
            pl.BlockSpec((C, C), lambda bb, it: (0, 0)),
            vecC, vecC, vecC,
            rows, rows, vecH, vecH,
        ],
        out_specs=(
            pl.BlockSpec((1, ti, N, _H), lambda bb, it: (bb, it, 0, 0)),
            rows,
        ),
        compiler_params=pltpu.CompilerParams(
            dimension_semantics=("parallel", "parallel"),
            vmem_limit_bytes=_VMEM_LIMIT),
    )(epk, b0, ahg0, w01, b01, se0, te0, b1, ahg1, wc1, bc1, se1, te1,
      hpre1, hres, sh1, th1)


def _bn_fold(stats, gamma_h, beta_h, gamma_e, beta_e, B, N, eps=1e-5):
    mh = float(B * N)
    h_mean = jnp.sum(stats[:, :, 0, :], axis=(0, 1)) / mh
    h_var = jnp.maximum(
        jnp.sum(stats[:, :, 1, :], axis=(0, 1)) / mh - h_mean * h_mean, 0.0)
    h_scale = gamma_h * jax.lax.rsqrt(h_var + eps)
    h_shift = beta_h - h_mean * h_scale
    me = float(B * N * N)
    e_mean = jnp.sum(stats[:, :, 2, :], axis=(0, 1)) / me
    e_var = jnp.maximum(
        jnp.sum(stats[:, :, 3, :], axis=(0, 1)) / me - e_mean * e_mean, 0.0)
    e_scale = gamma_e * jax.lax.rsqrt(e_var + eps)
    e_shift = beta_e - e_mean * e_scale
    return h_scale, h_shift, e_scale, e_shift


def kernel(nodes, edges,
           init_node_w, init_node_b, init_edge_w, init_edge_b,
           l0_U_w, l0_U_b, l0_V_w, l0_V_b, l0_A_w, l0_A_b,
           l0_B_w, l0_B_b, l0_C_w, l0_C_b,
           l0_norm_h_gamma, l0_norm_h_beta, l0_norm_e_gamma, l0_norm_e_beta,
           l1_U_w, l1_U_b, l1_V_w, l1_V_b, l1_A_w, l1_A_b,
           l1_B_w, l1_B_b, l1_C_w, l1_C_b,
           l1_norm_h_gamma, l1_norm_h_beta, l1_norm_e_gamma, l1_norm_e_beta):
    B, N, node_dim = nodes.shape
    H = _H
    P = 8
    G = N // P
    C = P * H
    ti = 32 if N % 32 == 0 else N
    f32 = jnp.float32

    eyeP = jnp.eye(P, dtype=f32)
    weP = jnp.kron(eyeP, init_edge_w)
    beP = jnp.tile(init_edge_b, P).reshape(1, C)
    wc0P = jnp.kron(eyeP, l0_C_w)
    bc0P = jnp.tile(l0_C_b, P).reshape(1, C)
    wc1P = jnp.kron(eyeP, l1_C_w)
    bc1P = jnp.tile(l1_C_b, P).reshape(1, C)
    wce0 = weP @ wc0P
    bce0 = beP @ wc0P + bc0P
    w01 = jnp.concatenate([weP, wce0], axis=1)
    b01 = jnp.concatenate([beP, bce0], axis=1)

    epk = edges.reshape(B, N * G, 2 * P)

    wu0 = jnp.concatenate([l0_U_w, l0_V_w, l0_A_w, l0_B_w], axis=1)
    bu0 = jnp.concatenate([l0_U_b, l0_V_b, l0_A_b, l0_B_b], axis=0)
    wu1 = jnp.concatenate([l1_U_w, l1_V_w, l1_A_w, l1_B_w], axis=1)
    bu1 = jnp.concatenate([l1_U_b, l1_V_b, l1_A_b, l1_B_b], axis=0)

    def pack_g(col):
        return col.reshape(B, G, C)

    h0_2d, uvab0 = _node_init(nodes.reshape(B * N, node_dim),
                              init_node_w, init_node_b, wu0, bu0)
    u0 = uvab0[:, 0:H].reshape(B, N, H)
    vhg0 = pack_g(uvab0[:, H:2 * H])
    ahg0 = pack_g(uvab0[:, 2 * H:3 * H])
    b0 = uvab0[:, 3 * H:4 * H].reshape(B, N, H)

    hpre0, st0 = _pass_a(epk, u0, b0, ahg0, vhg0, wce0, bce0,
                         ti=ti, G=G, P=P, N=N)
    hs0, ht0, es0, et0 = _bn_fold(st0, l0_norm_h_gamma, l0_norm_h_beta,
                                  l0_norm_e_gamma, l0_norm_e_beta, B, N)
    se0 = jnp.tile(es0, P).reshape(1, C)
    te0 = jnp.tile(et0, P).reshape(1, C)

    h1_2d, uvab1 = _node_update(hpre0.reshape(B * N, H), h0_2d,
                                hs0, ht0, wu1, bu1)
    u1 = uvab1[:, 0:H].reshape(B, N, H)
    vhg1 = pack_g(uvab1[:, H:2 * H])
    ahg1 = pack_g(uvab1[:, 2 * H:3 * H])
    b1 = uvab1[:, 3 * H:4 * H].reshape(B, N, H)

    hpre1, st1 = _pass_b(epk, b0, ahg0, w01, b01, se0, te0,
                         u1, b1, ahg1, vhg1, wc1P, bc1P,
                         ti=ti, G=G, P=P, N=N)
    hs1, ht1, es1, et1 = _bn_fold(st1, l1_norm_h_gamma, l1_norm_h_beta,
                                  l1_norm_e_gamma, l1_norm_e_beta, B, N)
    se1 = jnp.tile(es1, P).reshape(1, C)
    te1 = jnp.tile(et1, P).reshape(1, C)

    e_out, h_out = _pass_c(epk, b0, ahg0, w01, b01, se0, te0,
                           b1, ahg1, wc1P, bc1P, se1, te1,
                           hpre1, h1_2d.reshape(B, N, H),
                           hs1.reshape(1, H), ht1.reshape(1, H),
                           ti=ti, G=G, P=P, N=N)
    return h_out, e_out


# speedup vs baseline: 5.2311x; 2.1256x over previous
import functools

import jax
import jax.numpy as jnp
from jax.experimental import pallas as pl
from jax.experimental.pallas import tpu as pltpu

_VMEM_LIMIT = 48 * 1024 * 1024
_H = 32


def _node_init_body(x_ref, wn_ref, bn_ref, wu_ref, bu_ref, h_ref, uvab_ref):
    h = (jnp.dot(x_ref[...], wn_ref[...], preferred_element_type=jnp.float32)
         + bn_ref[...])
    h_ref[...] = h
    uvab_ref[...] = (
        jnp.dot(h, wu_ref[...], preferred_element_type=jnp.float32)
        + bu_ref[...])


def _node_init(x2d, wn, bn, wu, bu):
    m, k = x2d.shape
    tm = m // 8 if m % 8 == 0 else m
    return pl.pallas_call(
        _node_init_body,
        grid=(m // tm,),
        out_shape=(jax.ShapeDtypeStruct((m, _H), jnp.float32),
                   jax.ShapeDtypeStruct((m, 4 * _H), jnp.float32)),
        in_specs=[
            pl.BlockSpec((tm, k), lambda i: (i, 0)),
            pl.BlockSpec((k, _H), lambda i: (0, 0)),
            pl.BlockSpec((1, _H), lambda i: (0, 0)),
            pl.BlockSpec((_H, 4 * _H), lambda i: (0, 0)),
            pl.BlockSpec((1, 4 * _H), lambda i: (0, 0)),
        ],
        out_specs=(pl.BlockSpec((tm, _H), lambda i: (i, 0)),
                   pl.BlockSpec((tm, 4 * _H), lambda i: (i, 0))),
        compiler_params=pltpu.CompilerParams(
            dimension_semantics=("parallel",), vmem_limit_bytes=_VMEM_LIMIT),
    )(x2d, wn, bn.reshape(1, _H), wu, bu.reshape(1, 4 * _H))


def _node_update_body(hp_ref, hr_ref, s_ref, t_ref, wu_ref, bu_ref,
                      h_ref, uvab_ref):
    y = hp_ref[...] * s_ref[...] + t_ref[...]
    h = hr_ref[...] + jnp.maximum(y, 0.0)
    h_ref[...] = h
    uvab_ref[...] = (
        jnp.dot(h, wu_ref[...], preferred_element_type=jnp.float32)
        + bu_ref[...])


def _node_update(hpre2d, hres2d, scale, shift, wu, bu):
    m = hpre2d.shape[0]
    tm = m // 8 if m % 8 == 0 else m
    row = pl.BlockSpec((tm, _H), lambda i: (i, 0))
    vec = pl.BlockSpec((1, _H), lambda i: (0, 0))
    return pl.pallas_call(
        _node_update_body,
        grid=(m // tm,),
        out_shape=(jax.ShapeDtypeStruct((m, _H), jnp.float32),
                   jax.ShapeDtypeStruct((m, 4 * _H), jnp.float32)),
        in_specs=[row, row, vec, vec,
                  pl.BlockSpec((_H, 4 * _H), lambda i: (0, 0)),
                  pl.BlockSpec((1, 4 * _H), lambda i: (0, 0))],
        out_specs=(row, pl.BlockSpec((tm, 4 * _H), lambda i: (i, 0))),
        compiler_params=pltpu.CompilerParams(
            dimension_semantics=("parallel",), vmem_limit_bytes=_VMEM_LIMIT),
    )(hpre2d, hres2d, scale.reshape(1, _H), shift.reshape(1, _H),
      wu, bu.reshape(1, 4 * _H))


def _embed_edges_t(epk_ref, we, be, ti, N):
    x0 = epk_ref[0, :, 0, :]
    x1 = epk_ref[0, :, 1, :]
    w0 = we[0].reshape(1, _H, 1)
    w1 = we[1].reshape(1, _H, 1)
    return (x0[:, None, :] * w0 + x1[:, None, :] * w1
            + be.reshape(1, _H, 1))


def _cproj_t(e_t3, wbig_ref, bc, ti, N):
    e2 = e_t3.reshape(ti * _H, N)
    w = wbig_ref[...]
    parts = [
        jnp.dot(w, e2[s * 8 * _H:(s + 1) * 8 * _H, :],
                preferred_element_type=jnp.float32)
        for s in range(ti // 8)
    ]
    ce = jnp.concatenate(parts, axis=0) if len(parts) > 1 else parts[0]
    return ce.reshape(ti, _H, N) + bc.reshape(1, _H, 1)


def _gate_aggregate_t(e_new, vh_t, u, N):
    gated = jax.nn.sigmoid(e_new) * vh_t[None, :, :]
    return u + jnp.sum(gated, axis=2) * (1.0 / N)


def _write_stats_t(st_ref, h_pre, e_new):
    es = jnp.sum(e_new, axis=(0, 2)).reshape(1, _H)
    eq = jnp.sum(e_new * e_new, axis=(0, 2)).reshape(1, _H)
    st_ref[0, 0] = jnp.concatenate(
        [jnp.sum(h_pre, axis=0, keepdims=True),
         jnp.sum(h_pre * h_pre, axis=0, keepdims=True), es, eq], axis=0)


def _pass_a_body(epk_ref, u_ref, b_ref, aht_ref, vht_ref, wbig_ref,
                 we_ref, hpre_ref, st_ref, *, ti, N):
    e0 = _embed_edges_t(epk_ref, we_ref[0:2, :], we_ref[2, :], ti, N)
    ce = _cproj_t(e0, wbig_ref, we_ref[3, :], ti, N)
    e_new = ce + aht_ref[0][None, :, :] + b_ref[0][:, :, None]
    h_pre = _gate_aggregate_t(e_new, vht_ref[0], u_ref[0], N)
    hpre_ref[0] = h_pre
    _write_stats_t(st_ref, h_pre, e_new)


def _pass_a(epk, u, b, aht, vht, wbig, wepk, *, ti, N):
    B = epk.shape[0]
    n_it = N // ti
    body = functools.partial(_pass_a_body, ti=ti, N=N)
    rows = pl.BlockSpec((1, ti, _H), lambda bb, it: (bb, it, 0))
    slab = pl.BlockSpec((1, _H, N), lambda bb, it: (bb, 0, 0))
    return pl.pallas_call(
        body,
        grid=(B, n_it),
        out_shape=(jax.ShapeDtypeStruct((B, N, _H), jnp.float32),
                   jax.ShapeDtypeStruct((B, n_it, 4, _H), jnp.float32)),
        in_specs=[
            pl.BlockSpec((1, ti, 2, N), lambda bb, it: (bb, it, 0, 0)),
            rows, rows, slab, slab,
            pl.BlockSpec((8 * _H, 8 * _H), lambda bb, it: (0, 0)),
            pl.BlockSpec((4, _H), lambda bb, it: (0, 0)),
        ],
        out_specs=(
            rows,
            pl.BlockSpec((1, 1, 4, _H), lambda bb, it: (bb, it, 0, 0)),
        ),
        compiler_params=pltpu.CompilerParams(
            dimension_semantics=("parallel", "parallel"),
            vmem_limit_bytes=_VMEM_LIMIT),
    )(epk, u, b, aht, vht, wbig, wepk)


def _pass_b_body(epk_ref, b0_ref, aht0_ref, wbig0_ref, we_ref, se0_ref,
                 u1_ref, b1_ref, aht1_ref, vht1_ref, wbig1_ref, bc1_ref,
                 hpre_ref, st_ref, *, ti, N):
    e0 = _embed_edges_t(epk_ref, we_ref[0:2, :], we_ref[2, :], ti, N)
    ce0 = _cproj_t(e0, wbig0_ref, we_ref[3, :], ti, N)
    e_new0 = ce0 + aht0_ref[0][None, :, :] + b0_ref[0][:, :, None]
    s0 = se0_ref[0, :].reshape(1, _H, 1)
    t0 = se0_ref[1, :].reshape(1, _H, 1)
    e1 = e0 + jnp.maximum(e_new0 * s0 + t0, 0.0)
    ce1 = _cproj_t(e1, wbig1_ref, bc1_ref[0, :], ti, N)
    e_new1 = ce1 + aht1_ref[0][None, :, :] + b1_ref[0][:, :, None]
    h_pre = _gate_aggregate_t(e_new1, vht1_ref[0], u1_ref[0], N)
    hpre_ref[0] = h_pre
    _write_stats_t(st_ref, h_pre, e_new1)


def _pass_b(epk, b0, aht0, wbig0, wepk, se0, u1, b1, aht1, vht1, wbig1, bc1,
            *, ti, N):
    B = epk.shape[0]
    n_it = N // ti
    body = functools.partial(_pass_b_body, ti=ti, N=N)
    rows = pl.BlockSpec((1, ti, _H), lambda bb, it: (bb, it, 0))
    slab = pl.BlockSpec((1, _H, N), lambda bb, it: (bb, 0, 0))
    wfull = pl.BlockSpec((8 * _H, 8 * _H), lambda bb, it: (0, 0))
    return pl.pallas_call(
        body,
        grid=(B, n_it),
        out_shape=(jax.ShapeDtypeStruct((B, N, _H), jnp.float32),
                   jax.ShapeDtypeStruct((B, n_it, 4, _H), jnp.float32)),
        in_specs=[
            pl.BlockSpec((1, ti, 2, N), lambda bb, it: (bb, it, 0, 0)),
            rows, slab, wfull,
            pl.BlockSpec((4, _H), lambda bb, it: (0, 0)),
            pl.BlockSpec((2, _H), lambda bb, it: (0, 0)),
            rows, rows, slab, slab, wfull,
            pl.BlockSpec((1, _H), lambda bb, it: (0, 0)),
        ],
        out_specs=(
            rows,
            pl.BlockSpec((1, 1, 4, _H), lambda bb, it: (bb, it, 0, 0)),
        ),
        compiler_params=pltpu.CompilerParams(
            dimension_semantics=("parallel", "parallel"),
            vmem_limit_bytes=_VMEM_LIMIT),
    )(epk, b0, aht0, wbig0, wepk, se0, u1, b1, aht1, vht1, wbig1, bc1)


def _pass_c_body(epk_ref, b0_ref, aht0_ref, wbig0_ref, we_ref, se0_ref,
                 b1_ref, aht1_ref, wbig1_ref, bc1_ref, se1_ref,
                 hp1_ref, hr_ref, sh1_ref,
                 eout_ref, hout_ref, *, ti, N):
    e0 = _embed_edges_t(epk_ref, we_ref[0:2, :], we_ref[2, :], ti, N)
    ce0 = _cproj_t(e0, wbig0_ref, we_ref[3, :], ti, N)
    e_new0 = ce0 + aht0_ref[0][None, :, :] + b0_ref[0][:, :, None]
    s0 = se0_ref[0, :].reshape(1, _H, 1)
    t0 = se0_ref[1, :].reshape(1, _H, 1)
    e1 = e0 + jnp.maximum(e_new0 * s0 + t0, 0.0)
    ce1 = _cproj_t(e1, wbig1_ref, bc1_ref[0, :], ti, N)
    e_new1 = ce1 + aht1_ref[0][None, :, :] + b1_ref[0][:, :, None]
    s1 = se1_ref[0, :].reshape(1, _H, 1)
    t1 = se1_ref[1, :].reshape(1, _H, 1)
    eout_ref[0] = e1 + jnp.maximum(e_new1 * s1 + t1, 0.0)
    yh = hp1_ref[0] * sh1_ref[0:1, :] + sh1_ref[1:2, :]
    hout_ref[0] = hr_ref[0] + jnp.maximum(yh, 0.0)


def _pass_c(epk, b0, aht0, wbig0, wepk, se0, b1, aht1, wbig1, bc1, se1,
            hpre1, hres, sh1, *, ti, N):
    B = epk.shape[0]
    n_it = N // ti
    body = functools.partial(_pass_c_body, ti=ti, N=N)
    rows = pl.BlockSpec((1, ti, _H), lambda bb, it: (bb, it, 0))
    slab = pl.BlockSpec((1, _H, N), lambda bb, it: (bb, 0, 0))
    wfull = pl.BlockSpec((8 * _H, 8 * _H), lambda bb, it: (0, 0))
    return pl.pallas_call(
        body,
        grid=(B, n_it),
        out_shape=(jax.ShapeDtypeStruct((B, N, _H, N), jnp.float32),
                   jax.ShapeDtypeStruct((B, N, _H), jnp.float32)),
        in_specs=[
            pl.BlockSpec((1, ti, 2, N), lambda bb, it: (bb, it, 0, 0)),
            rows, slab, wfull,
            pl.BlockSpec((4, _H), lambda bb, it: (0, 0)),
            pl.BlockSpec((2, _H), lambda bb, it: (0, 0)),
            rows, slab, wfull,
            pl.BlockSpec((1, _H), lambda bb, it: (0, 0)),
            pl.BlockSpec((2, _H), lambda bb, it: (0, 0)),
            rows, rows,
            pl.BlockSpec((2, _H), lambda bb, it: (0, 0)),
        ],
        out_specs=(
            pl.BlockSpec((1, ti, _H, N), lambda bb, it: (bb, it, 0, 0)),
            rows,
        ),
        compiler_params=pltpu.CompilerParams(
            dimension_semantics=("parallel", "parallel"),
            vmem_limit_bytes=_VMEM_LIMIT),
    )(epk, b0, aht0, wbig0, wepk, se0, b1, aht1, wbig1, bc1, se1,
      hpre1, hres, sh1)


def _bn_fold(stats, gamma_h, beta_h, gamma_e, beta_e, B, N, eps=1e-5):
    mh = float(B * N)
    h_mean = jnp.sum(stats[:, :, 0, :], axis=(0, 1)) / mh
    h_var = jnp.maximum(
        jnp.sum(stats[:, :, 1, :], axis=(0, 1)) / mh - h_mean * h_mean, 0.0)
    h_scale = gamma_h * jax.lax.rsqrt(h_var + eps)
    h_shift = beta_h - h_mean * h_scale
    me = float(B * N * N)
    e_mean = jnp.sum(stats[:, :, 2, :], axis=(0, 1)) / me
    e_var = jnp.maximum(
        jnp.sum(stats[:, :, 3, :], axis=(0, 1)) / me - e_mean * e_mean, 0.0)
    e_scale = gamma_e * jax.lax.rsqrt(e_var + eps)
    e_shift = beta_e - e_mean * e_scale
    return h_scale, h_shift, e_scale, e_shift


def kernel(nodes, edges,
           init_node_w, init_node_b, init_edge_w, init_edge_b,
           l0_U_w, l0_U_b, l0_V_w, l0_V_b, l0_A_w, l0_A_b,
           l0_B_w, l0_B_b, l0_C_w, l0_C_b,
           l0_norm_h_gamma, l0_norm_h_beta, l0_norm_e_gamma, l0_norm_e_beta,
           l1_U_w, l1_U_b, l1_V_w, l1_V_b, l1_A_w, l1_A_b,
           l1_B_w, l1_B_b, l1_C_w, l1_C_b,
           l1_norm_h_gamma, l1_norm_h_beta, l1_norm_e_gamma, l1_norm_e_beta):
    B, N, node_dim = nodes.shape
    H = _H
    ti = 32
    f32 = jnp.float32

    epk = jnp.swapaxes(edges, 2, 3)

    eye8 = jnp.eye(8, dtype=f32)
    wbig0 = jnp.kron(eye8, l0_C_w.T)
    wbig1 = jnp.kron(eye8, l1_C_w.T)
    wepk = jnp.concatenate(
        [init_edge_w, init_edge_b.reshape(1, H), l0_C_b.reshape(1, H)], axis=0)

    wu0 = jnp.concatenate([l0_U_w, l0_V_w, l0_A_w, l0_B_w], axis=1)
    bu0 = jnp.concatenate([l0_U_b, l0_V_b, l0_A_b, l0_B_b], axis=0)
    wu1 = jnp.concatenate([l1_U_w, l1_V_w, l1_A_w, l1_B_w], axis=1)
    bu1 = jnp.concatenate([l1_U_b, l1_V_b, l1_A_b, l1_B_b], axis=0)

    def t_slab(col):
        return jnp.swapaxes(col.reshape(B, N, H), 1, 2)

    h0_2d, uvab0 = _node_init(nodes.reshape(B * N, node_dim),
                              init_node_w, init_node_b, wu0, bu0)
    u0 = uvab0[:, 0:H].reshape(B, N, H)
    vht0 = t_slab(uvab0[:, H:2 * H])
    aht0 = t_slab(uvab0[:, 2 * H:3 * H])
    b0 = uvab0[:, 3 * H:4 * H].reshape(B, N, H)

    hpre0, st0 = _pass_a(epk, u0, b0, aht0, vht0, wbig0, wepk, ti=ti, N=N)
    hs0, ht0, es0, et0 = _bn_fold(st0, l0_norm_h_gamma, l0_norm_h_beta,
                                  l0_norm_e_gamma, l0_norm_e_beta, B, N)
    se0 = jnp.stack([es0, et0], axis=0)

    h1_2d, uvab1 = _node_update(hpre0.reshape(B * N, H), h0_2d,
                                hs0, ht0, wu1, bu1)
    u1 = uvab1[:, 0:H].reshape(B, N, H)
    vht1 = t_slab(uvab1[:, H:2 * H])
    aht1 = t_slab(uvab1[:, 2 * H:3 * H])
    b1 = uvab1[:, 3 * H:4 * H].reshape(B, N, H)

    hpre1, st1 = _pass_b(epk, b0, aht0, wbig0, wepk, se0,
                         u1, b1, aht1, vht1, wbig1, l1_C_b.reshape(1, H),
                         ti=ti, N=N)
    hs1, ht1, es1, et1 = _bn_fold(st1, l1_norm_h_gamma, l1_norm_h_beta,
                                  l1_norm_e_gamma, l1_norm_e_beta, B, N)
    se1 = jnp.stack([es1, et1], axis=0)
    sh1 = jnp.stack([hs1, ht1], axis=0)

    e_out_t, h_out = _pass_c(epk, b0, aht0, wbig0, wepk, se0,
                             b1, aht1, wbig1, l1_C_b.reshape(1, H), se1,
                             hpre1, h1_2d.reshape(B, N, H), sh1,
                             ti=ti, N=N)
    return h_out, jnp.swapaxes(e_out_t, 2, 3)


# speedup vs baseline: 5.4032x; 1.0329x over previous
import functools

import jax
import jax.numpy as jnp
from jax.experimental import pallas as pl
from jax.experimental.pallas import tpu as pltpu

_VMEM_LIMIT = 48 * 1024 * 1024
_H = 32


def _node_init_body(x_ref, wn_ref, bn_ref, wu_ref, bu_ref, h_ref, uvab_ref):
    h = (jnp.dot(x_ref[...], wn_ref[...], preferred_element_type=jnp.float32)
         + bn_ref[...])
    h_ref[...] = h
    uvab_ref[...] = (
        jnp.dot(h, wu_ref[...], preferred_element_type=jnp.float32)
        + bu_ref[...])


def _node_init(x2d, wn, bn, wu, bu):
    m, k = x2d.shape
    tm = m // 8 if m % 8 == 0 else m
    return pl.pallas_call(
        _node_init_body,
        grid=(m // tm,),
        out_shape=(jax.ShapeDtypeStruct((m, _H), jnp.float32),
                   jax.ShapeDtypeStruct((m, 4 * _H), jnp.float32)),
        in_specs=[
            pl.BlockSpec((tm, k), lambda i: (i, 0)),
            pl.BlockSpec((k, _H), lambda i: (0, 0)),
            pl.BlockSpec((1, _H), lambda i: (0, 0)),
            pl.BlockSpec((_H, 4 * _H), lambda i: (0, 0)),
            pl.BlockSpec((1, 4 * _H), lambda i: (0, 0)),
        ],
        out_specs=(pl.BlockSpec((tm, _H), lambda i: (i, 0)),
                   pl.BlockSpec((tm, 4 * _H), lambda i: (i, 0))),
        compiler_params=pltpu.CompilerParams(
            dimension_semantics=("parallel",), vmem_limit_bytes=_VMEM_LIMIT),
    )(x2d, wn, bn.reshape(1, _H), wu, bu.reshape(1, 4 * _H))


def _node_update_body(hp_ref, hr_ref, s_ref, t_ref, wu_ref, bu_ref,
                      h_ref, uvab_ref):
    y = hp_ref[...] * s_ref[...] + t_ref[...]
    h = hr_ref[...] + jnp.maximum(y, 0.0)
    h_ref[...] = h
    uvab_ref[...] = (
        jnp.dot(h, wu_ref[...], preferred_element_type=jnp.float32)
        + bu_ref[...])


def _node_update(hpre2d, hres2d, scale, shift, wu, bu):
    m = hpre2d.shape[0]
    tm = m // 8 if m % 8 == 0 else m
    row = pl.BlockSpec((tm, _H), lambda i: (i, 0))
    vec = pl.BlockSpec((1, _H), lambda i: (0, 0))
    return pl.pallas_call(
        _node_update_body,
        grid=(m // tm,),
        out_shape=(jax.ShapeDtypeStruct((m, _H), jnp.float32),
                   jax.ShapeDtypeStruct((m, 4 * _H), jnp.float32)),
        in_specs=[row, row, vec, vec,
                  pl.BlockSpec((_H, 4 * _H), lambda i: (0, 0)),
                  pl.BlockSpec((1, 4 * _H), lambda i: (0, 0))],
        out_specs=(row, pl.BlockSpec((tm, 4 * _H), lambda i: (i, 0))),
        compiler_params=pltpu.CompilerParams(
            dimension_semantics=("parallel",), vmem_limit_bytes=_VMEM_LIMIT),
    )(hpre2d, hres2d, scale.reshape(1, _H), shift.reshape(1, _H),
      wu, bu.reshape(1, 4 * _H))


def _embed_edges_t(epk_ref, we, be, ti, N):
    x0 = epk_ref[0, :, 0, :]
    x1 = epk_ref[0, :, 1, :]
    w0 = we[0].reshape(1, _H, 1)
    w1 = we[1].reshape(1, _H, 1)
    return (x0[:, None, :] * w0 + x1[:, None, :] * w1
            + be.reshape(1, _H, 1))


def _cproj_t(e_t3, wbig_ref, bc, ti, N):
    e2 = e_t3.reshape(ti * _H, N)
    w = wbig_ref[...]
    parts = [
        jnp.dot(w, e2[s * 8 * _H:(s + 1) * 8 * _H, :],
                preferred_element_type=jnp.float32)
        for s in range(ti // 8)
    ]
    ce = jnp.concatenate(parts, axis=0) if len(parts) > 1 else parts[0]
    return ce.reshape(ti, _H, N) + bc.reshape(1, _H, 1)


def _gate_aggregate_t(e_new, vh_t, u, N):
    gated = jax.nn.sigmoid(e_new) * vh_t[None, :, :]
    return u + jnp.sum(gated, axis=2) * (1.0 / N)


def _write_stats_t(st_ref, h_pre, e_new):
    es = jnp.sum(e_new, axis=(0, 2)).reshape(1, _H)
    eq = jnp.sum(e_new * e_new, axis=(0, 2)).reshape(1, _H)
    st_ref[0, 0] = jnp.concatenate(
        [jnp.sum(h_pre, axis=0, keepdims=True),
         jnp.sum(h_pre * h_pre, axis=0, keepdims=True), es, eq], axis=0)


def _pass_a_body(epk_ref, u_ref, b_ref, aht_ref, vht_ref, wbig_ref,
                 we_ref, hpre_ref, st_ref, *, ti, N):
    e0 = _embed_edges_t(epk_ref, we_ref[0:2, :], we_ref[2, :], ti, N)
    ce = _cproj_t(e0, wbig_ref, we_ref[3, :], ti, N)
    e_new = ce + aht_ref[0][None, :, :] + b_ref[0][:, :, None]
    h_pre = _gate_aggregate_t(e_new, vht_ref[0], u_ref[0], N)
    hpre_ref[0] = h_pre
    _write_stats_t(st_ref, h_pre, e_new)


def _pass_a(epk, u, b, aht, vht, wbig, wepk, *, ti, N):
    B = epk.shape[0]
    n_it = N // ti
    body = functools.partial(_pass_a_body, ti=ti, N=N)
    rows = pl.BlockSpec((1, ti, _H), lambda bb, it: (bb, it, 0))
    slab = pl.BlockSpec((1, _H, N), lambda bb, it: (bb, 0, 0))
    return pl.pallas_call(
        body,
        grid=(B, n_it),
        out_shape=(jax.ShapeDtypeStruct((B, N, _H), jnp.float32),
                   jax.ShapeDtypeStruct((B, n_it, 4, _H), jnp.float32)),
        in_specs=[
            pl.BlockSpec((1, ti, 2, N), lambda bb, it: (bb, it, 0, 0)),
            rows, rows, slab, slab,
            pl.BlockSpec((8 * _H, 8 * _H), lambda bb, it: (0, 0)),
            pl.BlockSpec((4, _H), lambda bb, it: (0, 0)),
        ],
        out_specs=(
            rows,
            pl.BlockSpec((1, 1, 4, _H), lambda bb, it: (bb, it, 0, 0)),
        ),
        compiler_params=pltpu.CompilerParams(
            dimension_semantics=("parallel", "parallel"),
            vmem_limit_bytes=_VMEM_LIMIT),
    )(epk, u, b, aht, vht, wbig, wepk)


def _pass_b_body(epk_ref, b0s_ref, aht0s_ref, wbig0s_ref, we_ref, cb0_ref,
                 u1_ref, b1_ref, aht1_ref, vht1_ref, wbig1_ref, bc1_ref,
                 hpre_ref, st_ref, *, ti, N):
    e0 = _embed_edges_t(epk_ref, we_ref[0:2, :], we_ref[2, :], ti, N)
    ce0s = _cproj_t(e0, wbig0s_ref, cb0_ref[0, :], ti, N)
    e_new0s = ce0s + aht0s_ref[0][None, :, :] + b0s_ref[0][:, :, None]
    e1 = e0 + jnp.maximum(e_new0s, 0.0)
    ce1 = _cproj_t(e1, wbig1_ref, bc1_ref[0, :], ti, N)
    e_new1 = ce1 + aht1_ref[0][None, :, :] + b1_ref[0][:, :, None]
    h_pre = _gate_aggregate_t(e_new1, vht1_ref[0], u1_ref[0], N)
    hpre_ref[0] = h_pre
    _write_stats_t(st_ref, h_pre, e_new1)


def _pass_b(epk, b0s, aht0s, wbig0s, wepk, cb0, u1, b1, aht1, vht1, wbig1,
            bc1, *, ti, N):
    B = epk.shape[0]
    n_it = N // ti
    body = functools.partial(_pass_b_body, ti=ti, N=N)
    rows = pl.BlockSpec((1, ti, _H), lambda bb, it: (bb, it, 0))
    slab = pl.BlockSpec((1, _H, N), lambda bb, it: (bb, 0, 0))
    wfull = pl.BlockSpec((8 * _H, 8 * _H), lambda bb, it: (0, 0))
    return pl.pallas_call(
        body,
        grid=(B, n_it),
        out_shape=(jax.ShapeDtypeStruct((B, N, _H), jnp.float32),
                   jax.ShapeDtypeStruct((B, n_it, 4, _H), jnp.float32)),
        in_specs=[
            pl.BlockSpec((1, ti, 2, N), lambda bb, it: (bb, it, 0, 0)),
            rows, slab, wfull,
            pl.BlockSpec((4, _H), lambda bb, it: (0, 0)),
            pl.BlockSpec((1, _H), lambda bb, it: (0, 0)),
            rows, rows, slab, slab, wfull,
            pl.BlockSpec((1, _H), lambda bb, it: (0, 0)),
        ],
        out_specs=(
            rows,
            pl.BlockSpec((1, 1, 4, _H), lambda bb, it: (bb, it, 0, 0)),
        ),
        compiler_params=pltpu.CompilerParams(
            dimension_semantics=("parallel", "parallel"),
            vmem_limit_bytes=_VMEM_LIMIT),
    )(epk, b0s, aht0s, wbig0s, wepk, cb0, u1, b1, aht1, vht1, wbig1, bc1)


def _pass_c_body(epk_ref, b0s_ref, aht0s_ref, wbig0s_ref, we_ref, cb0_ref,
                 b1s_ref, aht1s_ref, wbig1s_ref, cb1_ref,
                 hp1_ref, hr_ref, sh1_ref,
                 eout_ref, hout_ref, *, ti, N):
    e0 = _embed_edges_t(epk_ref, we_ref[0:2, :], we_ref[2, :], ti, N)
    ce0s = _cproj_t(e0, wbig0s_ref, cb0_ref[0, :], ti, N)
    e_new0s = ce0s + aht0s_ref[0][None, :, :] + b0s_ref[0][:, :, None]
    e1 = e0 + jnp.maximum(e_new0s, 0.0)
    ce1s = _cproj_t(e1, wbig1s_ref, cb1_ref[0, :], ti, N)
    e_new1s = ce1s + aht1s_ref[0][None, :, :] + b1s_ref[0][:, :, None]
    eout_ref[0] = e1 + jnp.maximum(e_new1s, 0.0)
    yh = hp1_ref[0] * sh1_ref[0:1, :] + sh1_ref[1:2, :]
    hout_ref[0] = hr_ref[0] + jnp.maximum(yh, 0.0)


def _pass_c(epk, b0s, aht0s, wbig0s, wepk, cb0, b1s, aht1s, wbig1s, cb1,
            hpre1, hres, sh1, *, ti, N):
    B = epk.shape[0]
    n_it = N // ti
    body = functools.partial(_pass_c_body, ti=ti, N=N)
    rows = pl.BlockSpec((1, ti, _H), lambda bb, it: (bb, it, 0))
    slab = pl.BlockSpec((1, _H, N), lambda bb, it: (bb, 0, 0))
    wfull = pl.BlockSpec((8 * _H, 8 * _H), lambda bb, it: (0, 0))
    return pl.pallas_call(
        body,
        grid=(B, n_it),
        out_shape=(jax.ShapeDtypeStruct((B, N, _H, N), jnp.float32),
                   jax.ShapeDtypeStruct((B, N, _H), jnp.float32)),
        in_specs=[
            pl.BlockSpec((1, ti, 2, N), lambda bb, it: (bb, it, 0, 0)),
            rows, slab, wfull,
            pl.BlockSpec((4, _H), lambda bb, it: (0, 0)),
            pl.BlockSpec((1, _H), lambda bb, it: (0, 0)),
            rows, slab, wfull,
            pl.BlockSpec((1, _H), lambda bb, it: (0, 0)),
            rows, rows,
            pl.BlockSpec((2, _H), lambda bb, it: (0, 0)),
        ],
        out_specs=(
            pl.BlockSpec((1, ti, _H, N), lambda bb, it: (bb, it, 0, 0)),
            rows,
        ),
        compiler_params=pltpu.CompilerParams(
            dimension_semantics=("parallel", "parallel"),
            vmem_limit_bytes=_VMEM_LIMIT),
    )(epk, b0s, aht0s, wbig0s, wepk, cb0, b1s, aht1s, wbig1s, cb1,
      hpre1, hres, sh1)


def _bn_fold(stats, gamma_h, beta_h, gamma_e, beta_e, B, N, eps=1e-5):
    mh = float(B * N)
    h_mean = jnp.sum(stats[:, :, 0, :], axis=(0, 1)) / mh
    h_var = jnp.maximum(
        jnp.sum(stats[:, :, 1, :], axis=(0, 1)) / mh - h_mean * h_mean, 0.0)
    h_scale = gamma_h * jax.lax.rsqrt(h_var + eps)
    h_shift = beta_h - h_mean * h_scale
    me = float(B * N * N)
    e_mean = jnp.sum(stats[:, :, 2, :], axis=(0, 1)) / me
    e_var = jnp.maximum(
        jnp.sum(stats[:, :, 3, :], axis=(0, 1)) / me - e_mean * e_mean, 0.0)
    e_scale = gamma_e * jax.lax.rsqrt(e_var + eps)
    e_shift = beta_e - e_mean * e_scale
    return h_scale, h_shift, e_scale, e_shift


def kernel(nodes, edges,
           init_node_w, init_node_b, init_edge_w, init_edge_b,
           l0_U_w, l0_U_b, l0_V_w, l0_V_b, l0_A_w, l0_A_b,
           l0_B_w, l0_B_b, l0_C_w, l0_C_b,
           l0_norm_h_gamma, l0_norm_h_beta, l0_norm_e_gamma, l0_norm_e_beta,
           l1_U_w, l1_U_b, l1_V_w, l1_V_b, l1_A_w, l1_A_b,
           l1_B_w, l1_B_b, l1_C_w, l1_C_b,
           l1_norm_h_gamma, l1_norm_h_beta, l1_norm_e_gamma, l1_norm_e_beta):
    B, N, node_dim = nodes.shape
    H = _H
    ti = 32
    f32 = jnp.float32

    epk = jnp.swapaxes(edges, 2, 3)

    eye8 = jnp.eye(8, dtype=f32)
    wbig0 = jnp.kron(eye8, l0_C_w.T)
    wbig1 = jnp.kron(eye8, l1_C_w.T)
    wepk = jnp.concatenate(
        [init_edge_w, init_edge_b.reshape(1, H), l0_C_b.reshape(1, H)], axis=0)

    wu0 = jnp.concatenate([l0_U_w, l0_V_w, l0_A_w, l0_B_w], axis=1)
    bu0 = jnp.concatenate([l0_U_b, l0_V_b, l0_A_b, l0_B_b], axis=0)
    wu1 = jnp.concatenate([l1_U_w, l1_V_w, l1_A_w, l1_B_w], axis=1)
    bu1 = jnp.concatenate([l1_U_b, l1_V_b, l1_A_b, l1_B_b], axis=0)

    def t_slab(col):
        return jnp.swapaxes(col.reshape(B, N, H), 1, 2)

    h0_2d, uvab0 = _node_init(nodes.reshape(B * N, node_dim),
                              init_node_w, init_node_b, wu0, bu0)
    u0 = uvab0[:, 0:H].reshape(B, N, H)
    vht0 = t_slab(uvab0[:, H:2 * H])
    aht0 = t_slab(uvab0[:, 2 * H:3 * H])
    b0 = uvab0[:, 3 * H:4 * H].reshape(B, N, H)

    hpre0, st0 = _pass_a(epk, u0, b0, aht0, vht0, wbig0, wepk, ti=ti, N=N)
    hs0, ht0, es0, et0 = _bn_fold(st0, l0_norm_h_gamma, l0_norm_h_beta,
                                  l0_norm_e_gamma, l0_norm_e_beta, B, N)
    wbig0s = jnp.kron(eye8, (l0_C_w * es0[None, :]).T)
    cb0 = (l0_C_b * es0 + et0).reshape(1, H)
    aht0s = aht0 * es0[None, :, None]
    b0s = b0 * es0[None, None, :]

    h1_2d, uvab1 = _node_update(hpre0.reshape(B * N, H), h0_2d,
                                hs0, ht0, wu1, bu1)
    u1 = uvab1[:, 0:H].reshape(B, N, H)
    vht1 = t_slab(uvab1[:, H:2 * H])
    aht1 = t_slab(uvab1[:, 2 * H:3 * H])
    b1 = uvab1[:, 3 * H:4 * H].reshape(B, N, H)

    hpre1, st1 = _pass_b(epk, b0s, aht0s, wbig0s, wepk, cb0,
                         u1, b1, aht1, vht1, wbig1, l1_C_b.reshape(1, H),
                         ti=ti, N=N)
    hs1, ht1, es1, et1 = _bn_fold(st1, l1_norm_h_gamma, l1_norm_h_beta,
                                  l1_norm_e_gamma, l1_norm_e_beta, B, N)
    sh1 = jnp.stack([hs1, ht1], axis=0)
    wbig1s = jnp.kron(eye8, (l1_C_w * es1[None, :]).T)
    cb1 = (l1_C_b * es1 + et1).reshape(1, H)
    aht1s = aht1 * es1[None, :, None]
    b1s = b1 * es1[None, None, :]

    e_out_t, h_out = _pass_c(epk, b0s, aht0s, wbig0s, wepk, cb0,
                             b1s, aht1s, wbig1s, cb1,
                             hpre1, h1_2d.reshape(B, N, H), sh1,
                             ti=ti, N=N)
    return h_out, jnp.swapaxes(e_out_t, 2, 3)


# speedup vs baseline: 5.5236x; 1.0223x over previous
import functools

import jax
import jax.numpy as jnp
from jax.experimental import pallas as pl
from jax.experimental.pallas import tpu as pltpu

_VMEM_LIMIT = 48 * 1024 * 1024
_H = 32


def _node_init_body(x_ref, wn_ref, bn_ref, wu_ref, bu_ref, h_ref, uvab_ref):
    h = (jnp.dot(x_ref[...], wn_ref[...], preferred_element_type=jnp.float32)
         + bn_ref[...])
    h_ref[...] = h
    uvab_ref[...] = (
        jnp.dot(h, wu_ref[...], preferred_element_type=jnp.float32)
        + bu_ref[...])


def _node_init(x2d, wn, bn, wu, bu):
    m, k = x2d.shape
    tm = m // 8 if m % 8 == 0 else m
    return pl.pallas_call(
        _node_init_body,
        grid=(m // tm,),
        out_shape=(jax.ShapeDtypeStruct((m, _H), jnp.float32),
                   jax.ShapeDtypeStruct((m, 4 * _H), jnp.float32)),
        in_specs=[
            pl.BlockSpec((tm, k), lambda i: (i, 0)),
            pl.BlockSpec((k, _H), lambda i: (0, 0)),
            pl.BlockSpec((1, _H), lambda i: (0, 0)),
            pl.BlockSpec((_H, 4 * _H), lambda i: (0, 0)),
            pl.BlockSpec((1, 4 * _H), lambda i: (0, 0)),
        ],
        out_specs=(pl.BlockSpec((tm, _H), lambda i: (i, 0)),
                   pl.BlockSpec((tm, 4 * _H), lambda i: (i, 0))),
        compiler_params=pltpu.CompilerParams(
            dimension_semantics=("parallel",), vmem_limit_bytes=_VMEM_LIMIT),
    )(x2d, wn, bn.reshape(1, _H), wu, bu.reshape(1, 4 * _H))


def _node_update_body(hp_ref, hr_ref, s_ref, t_ref, wu_ref, bu_ref,
                      h_ref, uvab_ref):
    y = hp_ref[...] * s_ref[...] + t_ref[...]
    h = hr_ref[...] + jnp.maximum(y, 0.0)
    h_ref[...] = h
    uvab_ref[...] = (
        jnp.dot(h, wu_ref[...], preferred_element_type=jnp.float32)
        + bu_ref[...])


def _node_update(hpre2d, hres2d, scale, shift, wu, bu):
    m = hpre2d.shape[0]
    tm = m // 8 if m % 8 == 0 else m
    row = pl.BlockSpec((tm, _H), lambda i: (i, 0))
    vec = pl.BlockSpec((1, _H), lambda i: (0, 0))
    return pl.pallas_call(
        _node_update_body,
        grid=(m // tm,),
        out_shape=(jax.ShapeDtypeStruct((m, _H), jnp.float32),
                   jax.ShapeDtypeStruct((m, 4 * _H), jnp.float32)),
        in_specs=[row, row, vec, vec,
                  pl.BlockSpec((_H, 4 * _H), lambda i: (0, 0)),
                  pl.BlockSpec((1, 4 * _H), lambda i: (0, 0))],
        out_specs=(row, pl.BlockSpec((tm, 4 * _H), lambda i: (i, 0))),
        compiler_params=pltpu.CompilerParams(
            dimension_semantics=("parallel",), vmem_limit_bytes=_VMEM_LIMIT),
    )(hpre2d, hres2d, scale.reshape(1, _H), shift.reshape(1, _H),
      wu, bu.reshape(1, 4 * _H))


def _embed_edges_t(epk_ref, we, be, ti, N):
    x0 = epk_ref[0, :, 0, :]
    x1 = epk_ref[0, :, 1, :]
    w0 = we[0].reshape(1, _H, 1)
    w1 = we[1].reshape(1, _H, 1)
    return (x0[:, None, :] * w0 + x1[:, None, :] * w1
            + be.reshape(1, _H, 1))


def _cproj_t(e_t3, wbig_ref, ti, N):
    e2 = e_t3.reshape(ti * _H, N)
    w = wbig_ref[...]
    parts = [
        jnp.dot(w, e2[s * 8 * _H:(s + 1) * 8 * _H, :],
                preferred_element_type=jnp.float32)
        for s in range(ti // 8)
    ]
    ce = jnp.concatenate(parts, axis=0) if len(parts) > 1 else parts[0]
    return ce.reshape(ti, _H, N)


def _gate_aggregate_t(e_new, vh_t, u, N):
    gated = jax.nn.sigmoid(e_new) * vh_t[None, :, :]
    return u + jnp.sum(gated, axis=2) * (1.0 / N)


def _write_stats_t(st_ref, h_pre, e_new):
    es = jnp.sum(e_new, axis=(0, 2)).reshape(1, _H)
    eq = jnp.sum(e_new * e_new, axis=(0, 2)).reshape(1, _H)
    st_ref[0, 0] = jnp.concatenate(
        [jnp.sum(h_pre, axis=0, keepdims=True),
         jnp.sum(h_pre * h_pre, axis=0, keepdims=True), es, eq], axis=0)


def _pass_a_body(epk_ref, u_ref, b_ref, aht_ref, vht_ref, wbig_ref,
                 we_ref, hpre_ref, st_ref, *, ti, N):
    e0 = _embed_edges_t(epk_ref, we_ref[0:2, :], we_ref[2, :], ti, N)
    ce = _cproj_t(e0, wbig_ref, ti, N)
    e_new = ce + aht_ref[0][None, :, :] + b_ref[0][:, :, None]
    h_pre = _gate_aggregate_t(e_new, vht_ref[0], u_ref[0], N)
    hpre_ref[0] = h_pre
    _write_stats_t(st_ref, h_pre, e_new)


def _pass_a(epk, u, b, aht, vht, wbig, wepk, *, ti, N):
    B = epk.shape[0]
    n_it = N // ti
    body = functools.partial(_pass_a_body, ti=ti, N=N)
    rows = pl.BlockSpec((1, ti, _H), lambda bb, it: (bb, it, 0))
    slab = pl.BlockSpec((1, _H, N), lambda bb, it: (bb, 0, 0))
    return pl.pallas_call(
        body,
        grid=(B, n_it),
        out_shape=(jax.ShapeDtypeStruct((B, N, _H), jnp.float32),
                   jax.ShapeDtypeStruct((B, n_it, 4, _H), jnp.float32)),
        in_specs=[
            pl.BlockSpec((1, ti, 2, N), lambda bb, it: (bb, it, 0, 0)),
            rows, rows, slab, slab,
            pl.BlockSpec((8 * _H, 8 * _H), lambda bb, it: (0, 0)),
            pl.BlockSpec((4, _H), lambda bb, it: (0, 0)),
        ],
        out_specs=(
            rows,
            pl.BlockSpec((1, 1, 4, _H), lambda bb, it: (bb, it, 0, 0)),
        ),
        compiler_params=pltpu.CompilerParams(
            dimension_semantics=("parallel", "parallel"),
            vmem_limit_bytes=_VMEM_LIMIT),
    )(epk, u, b, aht, vht, wbig, wepk)


def _pass_b_body(epk_ref, b0s_ref, aht0s_ref, wbig0s_ref, we_ref,
                 u1_ref, b1_ref, aht1_ref, vht1_ref, wbig1_ref,
                 hpre_ref, st_ref, *, ti, N):
    e0 = _embed_edges_t(epk_ref, we_ref[0:2, :], we_ref[2, :], ti, N)
    ce0s = _cproj_t(e0, wbig0s_ref, ti, N)
    e_new0s = ce0s + aht0s_ref[0][None, :, :] + b0s_ref[0][:, :, None]
    e1 = e0 + jnp.maximum(e_new0s, 0.0)
    ce1 = _cproj_t(e1, wbig1_ref, ti, N)
    e_new1 = ce1 + aht1_ref[0][None, :, :] + b1_ref[0][:, :, None]
    h_pre = _gate_aggregate_t(e_new1, vht1_ref[0], u1_ref[0], N)
    hpre_ref[0] = h_pre
    _write_stats_t(st_ref, h_pre, e_new1)


def _pass_b(epk, b0s, aht0s, wbig0s, wepk, u1, b1, aht1, vht1, wbig1,
            *, ti, N):
    B = epk.shape[0]
    n_it = N // ti
    body = functools.partial(_pass_b_body, ti=ti, N=N)
    rows = pl.BlockSpec((1, ti, _H), lambda bb, it: (bb, it, 0))
    slab = pl.BlockSpec((1, _H, N), lambda bb, it: (bb, 0, 0))
    wfull = pl.BlockSpec((8 * _H, 8 * _H), lambda bb, it: (0, 0))
    return pl.pallas_call(
        body,
        grid=(B, n_it),
        out_shape=(jax.ShapeDtypeStruct((B, N, _H), jnp.float32),
                   jax.ShapeDtypeStruct((B, n_it, 4, _H), jnp.float32)),
        in_specs=[
            pl.BlockSpec((1, ti, 2, N), lambda bb, it: (bb, it, 0, 0)),
            rows, slab, wfull,
            pl.BlockSpec((4, _H), lambda bb, it: (0, 0)),
            rows, rows, slab, slab, wfull,
        ],
        out_specs=(
            rows,
            pl.BlockSpec((1, 1, 4, _H), lambda bb, it: (bb, it, 0, 0)),
        ),
        compiler_params=pltpu.CompilerParams(
            dimension_semantics=("parallel", "parallel"),
            vmem_limit_bytes=_VMEM_LIMIT),
    )(epk, b0s, aht0s, wbig0s, wepk, u1, b1, aht1, vht1, wbig1)


def _pass_c_body(epk_ref, b0s_ref, aht0s_ref, wbig0s_ref, we_ref,
                 b1s_ref, aht1s_ref, wbig1s_ref,
                 hp1_ref, hr_ref, sh1_ref,
                 eout_ref, hout_ref, *, ti, N):
    e0 = _embed_edges_t(epk_ref, we_ref[0:2, :], we_ref[2, :], ti, N)
    ce0s = _cproj_t(e0, wbig0s_ref, ti, N)
    e_new0s = ce0s + aht0s_ref[0][None, :, :] + b0s_ref[0][:, :, None]
    e1 = e0 + jnp.maximum(e_new0s, 0.0)
    ce1s = _cproj_t(e1, wbig1s_ref, ti, N)
    e_new1s = ce1s + aht1s_ref[0][None, :, :] + b1s_ref[0][:, :, None]
    eout_ref[0] = e1 + jnp.maximum(e_new1s, 0.0)
    yh = hp1_ref[0] * sh1_ref[0:1, :] + sh1_ref[1:2, :]
    hout_ref[0] = hr_ref[0] + jnp.maximum(yh, 0.0)


def _pass_c(epk, b0s, aht0s, wbig0s, wepk, b1s, aht1s, wbig1s,
            hpre1, hres, sh1, *, ti, N):
    B = epk.shape[0]
    n_it = N // ti
    body = functools.partial(_pass_c_body, ti=ti, N=N)
    rows = pl.BlockSpec((1, ti, _H), lambda bb, it: (bb, it, 0))
    slab = pl.BlockSpec((1, _H, N), lambda bb, it: (bb, 0, 0))
    wfull = pl.BlockSpec((8 * _H, 8 * _H), lambda bb, it: (0, 0))
    return pl.pallas_call(
        body,
        grid=(B, n_it),
        out_shape=(jax.ShapeDtypeStruct((B, N, _H, N), jnp.float32),
                   jax.ShapeDtypeStruct((B, N, _H), jnp.float32)),
        in_specs=[
            pl.BlockSpec((1, ti, 2, N), lambda bb, it: (bb, it, 0, 0)),
            rows, slab, wfull,
            pl.BlockSpec((4, _H), lambda bb, it: (0, 0)),
            rows, slab, wfull,
            rows, rows,
            pl.BlockSpec((2, _H), lambda bb, it: (0, 0)),
        ],
        out_specs=(
            pl.BlockSpec((1, ti, _H, N), lambda bb, it: (bb, it, 0, 0)),
            rows,
        ),
        compiler_params=pltpu.CompilerParams(
            dimension_semantics=("parallel", "parallel"),
            vmem_limit_bytes=_VMEM_LIMIT),
    )(epk, b0s, aht0s, wbig0s, wepk, b1s, aht1s, wbig1s,
      hpre1, hres, sh1)


def _bn_fold(stats, gamma_h, beta_h, gamma_e, beta_e, B, N, eps=1e-5):
    mh = float(B * N)
    h_mean = jnp.sum(stats[:, :, 0, :], axis=(0, 1)) / mh
    h_var = jnp.maximum(
        jnp.sum(stats[:, :, 1, :], axis=(0, 1)) / mh - h_mean * h_mean, 0.0)
    h_scale = gamma_h * jax.lax.rsqrt(h_var + eps)
    h_shift = beta_h - h_mean * h_scale
    me = float(B * N * N)
    e_mean = jnp.sum(stats[:, :, 2, :], axis=(0, 1)) / me
    e_var = jnp.maximum(
        jnp.sum(stats[:, :, 3, :], axis=(0, 1)) / me - e_mean * e_mean, 0.0)
    e_scale = gamma_e * jax.lax.rsqrt(e_var + eps)
    e_shift = beta_e - e_mean * e_scale
    return h_scale, h_shift, e_scale, e_shift


def kernel(nodes, edges,
           init_node_w, init_node_b, init_edge_w, init_edge_b,
           l0_U_w, l0_U_b, l0_V_w, l0_V_b, l0_A_w, l0_A_b,
           l0_B_w, l0_B_b, l0_C_w, l0_C_b,
           l0_norm_h_gamma, l0_norm_h_beta, l0_norm_e_gamma, l0_norm_e_beta,
           l1_U_w, l1_U_b, l1_V_w, l1_V_b, l1_A_w, l1_A_b,
           l1_B_w, l1_B_b, l1_C_w, l1_C_b,
           l1_norm_h_gamma, l1_norm_h_beta, l1_norm_e_gamma, l1_norm_e_beta):
    B, N, node_dim = nodes.shape
    H = _H
    ti = 32
    f32 = jnp.float32

    epk = jnp.swapaxes(edges, 2, 3)

    eye8 = jnp.eye(8, dtype=f32)
    wbig0 = jnp.kron(eye8, l0_C_w.T)
    wbig1 = jnp.kron(eye8, l1_C_w.T)
    wepk = jnp.concatenate(
        [init_edge_w, init_edge_b.reshape(1, H), l0_C_b.reshape(1, H)], axis=0)

    wu0 = jnp.concatenate([l0_U_w, l0_V_w, l0_A_w, l0_B_w], axis=1)
    bu0 = jnp.concatenate([l0_U_b, l0_V_b, l0_A_b, l0_B_b], axis=0)
    wu1 = jnp.concatenate([l1_U_w, l1_V_w, l1_A_w, l1_B_w], axis=1)
    bu1 = jnp.concatenate([l1_U_b, l1_V_b, l1_A_b, l1_B_b], axis=0)

    def t_slab(col):
        return jnp.swapaxes(col.reshape(B, N, H), 1, 2)

    h0_2d, uvab0 = _node_init(nodes.reshape(B * N, node_dim),
                              init_node_w, init_node_b, wu0, bu0)
    u0 = uvab0[:, 0:H].reshape(B, N, H)
    vht0 = t_slab(uvab0[:, H:2 * H])
    aht0 = t_slab(uvab0[:, 2 * H:3 * H]) + l0_C_b.reshape(1, H, 1)
    b0 = uvab0[:, 3 * H:4 * H].reshape(B, N, H)

    hpre0, st0 = _pass_a(epk, u0, b0, aht0, vht0, wbig0, wepk, ti=ti, N=N)
    hs0, ht0, es0, et0 = _bn_fold(st0, l0_norm_h_gamma, l0_norm_h_beta,
                                  l0_norm_e_gamma, l0_norm_e_beta, B, N)
    wbig0s = jnp.kron(eye8, (l0_C_w * es0[None, :]).T)
    aht0s = aht0 * es0[None, :, None] + et0.reshape(1, H, 1)
    b0s = b0 * es0[None, None, :]

    h1_2d, uvab1 = _node_update(hpre0.reshape(B * N, H), h0_2d,
                                hs0, ht0, wu1, bu1)
    u1 = uvab1[:, 0:H].reshape(B, N, H)
    vht1 = t_slab(uvab1[:, H:2 * H])
    aht1 = t_slab(uvab1[:, 2 * H:3 * H]) + l1_C_b.reshape(1, H, 1)
    b1 = uvab1[:, 3 * H:4 * H].reshape(B, N, H)

    hpre1, st1 = _pass_b(epk, b0s, aht0s, wbig0s, wepk,
                         u1, b1, aht1, vht1, wbig1, ti=ti, N=N)
    hs1, ht1, es1, et1 = _bn_fold(st1, l1_norm_h_gamma, l1_norm_h_beta,
                                  l1_norm_e_gamma, l1_norm_e_beta, B, N)
    sh1 = jnp.stack([hs1, ht1], axis=0)
    wbig1s = jnp.kron(eye8, (l1_C_w * es1[None, :]).T)
    aht1s = aht1 * es1[None, :, None] + et1.reshape(1, H, 1)
    b1s = b1 * es1[None, None, :]

    e_out_t, h_out = _pass_c(epk, b0s, aht0s, wbig0s, wepk,
                             b1s, aht1s, wbig1s,
                             hpre1, h1_2d.reshape(B, N, H), sh1,
                             ti=ti, N=N)
    return h_out, jnp.swapaxes(e_out_t, 2, 3)


# speedup vs baseline: 5.5466x; 1.0042x over previous
import functools

import jax
import jax.numpy as jnp
from jax.experimental import pallas as pl
from jax.experimental.pallas import tpu as pltpu

_VMEM_LIMIT = 48 * 1024 * 1024
_H = 32


def _node_init_body(x_ref, wn_ref, bn_ref, wu_ref, bu_ref, h_ref, uvab_ref):
    h = (jnp.dot(x_ref[...], wn_ref[...], preferred_element_type=jnp.float32)
         + bn_ref[...])
    h_ref[...] = h
    uvab_ref[...] = (
        jnp.dot(h, wu_ref[...], preferred_element_type=jnp.float32)
        + bu_ref[...])


def _node_init(x2d, wn, bn, wu, bu):
    m, k = x2d.shape
    tm = m // 8 if m % 8 == 0 else m
    return pl.pallas_call(
        _node_init_body,
        grid=(m // tm,),
        out_shape=(jax.ShapeDtypeStruct((m, _H), jnp.float32),
                   jax.ShapeDtypeStruct((m, 4 * _H), jnp.float32)),
        in_specs=[
            pl.BlockSpec((tm, k), lambda i: (i, 0)),
            pl.BlockSpec((k, _H), lambda i: (0, 0)),
            pl.BlockSpec((1, _H), lambda i: (0, 0)),
            pl.BlockSpec((_H, 4 * _H), lambda i: (0, 0)),
            pl.BlockSpec((1, 4 * _H), lambda i: (0, 0)),
        ],
        out_specs=(pl.BlockSpec((tm, _H), lambda i: (i, 0)),
                   pl.BlockSpec((tm, 4 * _H), lambda i: (i, 0))),
        compiler_params=pltpu.CompilerParams(
            dimension_semantics=("parallel",), vmem_limit_bytes=_VMEM_LIMIT),
    )(x2d, wn, bn.reshape(1, _H), wu, bu.reshape(1, 4 * _H))


def _node_update_body(hp_ref, hr_ref, s_ref, t_ref, wu_ref, bu_ref,
                      h_ref, uvab_ref):
    y = hp_ref[...] * s_ref[...] + t_ref[...]
    h = hr_ref[...] + jnp.maximum(y, 0.0)
    h_ref[...] = h
    uvab_ref[...] = (
        jnp.dot(h, wu_ref[...], preferred_element_type=jnp.float32)
        + bu_ref[...])


def _node_update(hpre2d, hres2d, scale, shift, wu, bu):
    m = hpre2d.shape[0]
    tm = m // 8 if m % 8 == 0 else m
    row = pl.BlockSpec((tm, _H), lambda i: (i, 0))
    vec = pl.BlockSpec((1, _H), lambda i: (0, 0))
    return pl.pallas_call(
        _node_update_body,
        grid=(m // tm,),
        out_shape=(jax.ShapeDtypeStruct((m, _H), jnp.float32),
                   jax.ShapeDtypeStruct((m, 4 * _H), jnp.float32)),
        in_specs=[row, row, vec, vec,
                  pl.BlockSpec((_H, 4 * _H), lambda i: (0, 0)),
                  pl.BlockSpec((1, 4 * _H), lambda i: (0, 0))],
        out_specs=(row, pl.BlockSpec((tm, 4 * _H), lambda i: (i, 0))),
        compiler_params=pltpu.CompilerParams(
            dimension_semantics=("parallel",), vmem_limit_bytes=_VMEM_LIMIT),
    )(hpre2d, hres2d, scale.reshape(1, _H), shift.reshape(1, _H),
      wu, bu.reshape(1, 4 * _H))


def _embed_edges_t(epk_ref, we, be, ti, N):
    x0 = epk_ref[0, :, 0, :]
    x1 = epk_ref[0, :, 1, :]
    w0 = we[0].reshape(1, _H, 1)
    w1 = we[1].reshape(1, _H, 1)
    return (x0[:, None, :] * w0 + x1[:, None, :] * w1
            + be.reshape(1, _H, 1))


def _cproj_t(e_t3, wbig_ref, ti, N):
    e2 = e_t3.reshape(ti * _H, N).astype(jnp.bfloat16)
    w = wbig_ref[...]
    parts = [
        jnp.dot(w, e2[s * 8 * _H:(s + 1) * 8 * _H, :],
                preferred_element_type=jnp.float32)
        for s in range(ti // 8)
    ]
    ce = jnp.concatenate(parts, axis=0) if len(parts) > 1 else parts[0]
    return ce.reshape(ti, _H, N)


def _gate_aggregate_t(e_new, vh_t, u, N):
    gated = jax.nn.sigmoid(e_new) * vh_t[None, :, :]
    return u + jnp.sum(gated, axis=2) * (1.0 / N)


def _write_stats_t(st_ref, h_pre, e_new):
    es = jnp.sum(e_new, axis=(0, 2)).reshape(1, _H)
    eq = jnp.sum(e_new * e_new, axis=(0, 2)).reshape(1, _H)
    st_ref[0, 0] = jnp.concatenate(
        [jnp.sum(h_pre, axis=0, keepdims=True),
         jnp.sum(h_pre * h_pre, axis=0, keepdims=True), es, eq], axis=0)


def _pass_a_body(epk_ref, u_ref, b_ref, aht_ref, vht_ref, wbig_ref,
                 we_ref, hpre_ref, st_ref, *, ti, N):
    e0 = _embed_edges_t(epk_ref, we_ref[0:2, :], we_ref[2, :], ti, N)
    ce = _cproj_t(e0, wbig_ref, ti, N)
    e_new = ce + aht_ref[0][None, :, :] + b_ref[0][:, :, None]
    h_pre = _gate_aggregate_t(e_new, vht_ref[0], u_ref[0], N)
    hpre_ref[0] = h_pre
    _write_stats_t(st_ref, h_pre, e_new)


def _pass_a(epk, u, b, aht, vht, wbig, wepk, *, ti, N):
    B = epk.shape[0]
    n_it = N // ti
    body = functools.partial(_pass_a_body, ti=ti, N=N)
    rows = pl.BlockSpec((1, ti, _H), lambda bb, it: (bb, it, 0))
    slab = pl.BlockSpec((1, _H, N), lambda bb, it: (bb, 0, 0))
    return pl.pallas_call(
        body,
        grid=(B, n_it),
        out_shape=(jax.ShapeDtypeStruct((B, N, _H), jnp.float32),
                   jax.ShapeDtypeStruct((B, n_it, 4, _H), jnp.float32)),
        in_specs=[
            pl.BlockSpec((1, ti, 2, N), lambda bb, it: (bb, it, 0, 0)),
            rows, rows, slab, slab,
            pl.BlockSpec((8 * _H, 8 * _H), lambda bb, it: (0, 0)),
            pl.BlockSpec((4, _H), lambda bb, it: (0, 0)),
        ],
        out_specs=(
            rows,
            pl.BlockSpec((1, 1, 4, _H), lambda bb, it: (bb, it, 0, 0)),
        ),
        compiler_params=pltpu.CompilerParams(
            dimension_semantics=("parallel", "parallel"),
            vmem_limit_bytes=_VMEM_LIMIT),
    )(epk, u, b, aht, vht, wbig, wepk)


def _pass_b_body(epk_ref, b0s_ref, aht0s_ref, wbig0s_ref, we_ref,
                 u1_ref, b1_ref, aht1_ref, vht1_ref, wbig1_ref,
                 hpre_ref, st_ref, *, ti, N):
    e0 = _embed_edges_t(epk_ref, we_ref[0:2, :], we_ref[2, :], ti, N)
    ce0s = _cproj_t(e0, wbig0s_ref, ti, N)
    e_new0s = ce0s + aht0s_ref[0][None, :, :] + b0s_ref[0][:, :, None]
    e1 = e0 + jnp.maximum(e_new0s, 0.0)
    ce1 = _cproj_t(e1, wbig1_ref, ti, N)
    e_new1 = ce1 + aht1_ref[0][None, :, :] + b1_ref[0][:, :, None]
    h_pre = _gate_aggregate_t(e_new1, vht1_ref[0], u1_ref[0], N)
    hpre_ref[0] = h_pre
    _write_stats_t(st_ref, h_pre, e_new1)


def _pass_b(epk, b0s, aht0s, wbig0s, wepk, u1, b1, aht1, vht1, wbig1,
            *, ti, N):
    B = epk.shape[0]
    n_it = N // ti
    body = functools.partial(_pass_b_body, ti=ti, N=N)
    rows = pl.BlockSpec((1, ti, _H), lambda bb, it: (bb, it, 0))
    slab = pl.BlockSpec((1, _H, N), lambda bb, it: (bb, 0, 0))
    wfull = pl.BlockSpec((8 * _H, 8 * _H), lambda bb, it: (0, 0))
    return pl.pallas_call(
        body,
        grid=(B, n_it),
        out_shape=(jax.ShapeDtypeStruct((B, N, _H), jnp.float32),
                   jax.ShapeDtypeStruct((B, n_it, 4, _H), jnp.float32)),
        in_specs=[
            pl.BlockSpec((1, ti, 2, N), lambda bb, it: (bb, it, 0, 0)),
            rows, slab, wfull,
            pl.BlockSpec((4, _H), lambda bb, it: (0, 0)),
            rows, rows, slab, slab, wfull,
        ],
        out_specs=(
            rows,
            pl.BlockSpec((1, 1, 4, _H), lambda bb, it: (bb, it, 0, 0)),
        ),
        compiler_params=pltpu.CompilerParams(
            dimension_semantics=("parallel", "parallel"),
            vmem_limit_bytes=_VMEM_LIMIT),
    )(epk, b0s, aht0s, wbig0s, wepk, u1, b1, aht1, vht1, wbig1)


def _pass_c_body(epk_ref, b0s_ref, aht0s_ref, wbig0s_ref, we_ref,
                 b1s_ref, aht1s_ref, wbig1s_ref,
                 hp1_ref, hr_ref, sh1_ref,
                 eout_ref, hout_ref, *, ti, N):
    e0 = _embed_edges_t(epk_ref, we_ref[0:2, :], we_ref[2, :], ti, N)
    ce0s = _cproj_t(e0, wbig0s_ref, ti, N)
    e_new0s = ce0s + aht0s_ref[0][None, :, :] + b0s_ref[0][:, :, None]
    e1 = e0 + jnp.maximum(e_new0s, 0.0)
    ce1s = _cproj_t(e1, wbig1s_ref, ti, N)
    e_new1s = ce1s + aht1s_ref[0][None, :, :] + b1s_ref[0][:, :, None]
    eout_ref[0] = e1 + jnp.maximum(e_new1s, 0.0)
    yh = hp1_ref[0] * sh1_ref[0:1, :] + sh1_ref[1:2, :]
    hout_ref[0] = hr_ref[0] + jnp.maximum(yh, 0.0)


def _pass_c(epk, b0s, aht0s, wbig0s, wepk, b1s, aht1s, wbig1s,
            hpre1, hres, sh1, *, ti, N):
    B = epk.shape[0]
    n_it = N // ti
    body = functools.partial(_pass_c_body, ti=ti, N=N)
    rows = pl.BlockSpec((1, ti, _H), lambda bb, it: (bb, it, 0))
    slab = pl.BlockSpec((1, _H, N), lambda bb, it: (bb, 0, 0))
    wfull = pl.BlockSpec((8 * _H, 8 * _H), lambda bb, it: (0, 0))
    return pl.pallas_call(
        body,
        grid=(B, n_it),
        out_shape=(jax.ShapeDtypeStruct((B, N, _H, N), jnp.float32),
                   jax.ShapeDtypeStruct((B, N, _H), jnp.float32)),
        in_specs=[
            pl.BlockSpec((1, ti, 2, N), lambda bb, it: (bb, it, 0, 0)),
            rows, slab, wfull,
            pl.BlockSpec((4, _H), lambda bb, it: (0, 0)),
            rows, slab, wfull,
            rows, rows,
            pl.BlockSpec((2, _H), lambda bb, it: (0, 0)),
        ],
        out_specs=(
            pl.BlockSpec((1, ti, _H, N), lambda bb, it: (bb, it, 0, 0)),
            rows,
        ),
        compiler_params=pltpu.CompilerParams(
            dimension_semantics=("parallel", "parallel"),
            vmem_limit_bytes=_VMEM_LIMIT),
    )(epk, b0s, aht0s, wbig0s, wepk, b1s, aht1s, wbig1s,
      hpre1, hres, sh1)


def _bn_fold(stats, gamma_h, beta_h, gamma_e, beta_e, B, N, eps=1e-5):
    mh = float(B * N)
    h_mean = jnp.sum(stats[:, :, 0, :], axis=(0, 1)) / mh
    h_var = jnp.maximum(
        jnp.sum(stats[:, :, 1, :], axis=(0, 1)) / mh - h_mean * h_mean, 0.0)
    h_scale = gamma_h * jax.lax.rsqrt(h_var + eps)
    h_shift = beta_h - h_mean * h_scale
    me = float(B * N * N)
    e_mean = jnp.sum(stats[:, :, 2, :], axis=(0, 1)) / me
    e_var = jnp.maximum(
        jnp.sum(stats[:, :, 3, :], axis=(0, 1)) / me - e_mean * e_mean, 0.0)
    e_scale = gamma_e * jax.lax.rsqrt(e_var + eps)
    e_shift = beta_e - e_mean * e_scale
    return h_scale, h_shift, e_scale, e_shift


def kernel(nodes, edges,
           init_node_w, init_node_b, init_edge_w, init_edge_b,
           l0_U_w, l0_U_b, l0_V_w, l0_V_b, l0_A_w, l0_A_b,
           l0_B_w, l0_B_b, l0_C_w, l0_C_b,
           l0_norm_h_gamma, l0_norm_h_beta, l0_norm_e_gamma, l0_norm_e_beta,
           l1_U_w, l1_U_b, l1_V_w, l1_V_b, l1_A_w, l1_A_b,
           l1_B_w, l1_B_b, l1_C_w, l1_C_b,
           l1_norm_h_gamma, l1_norm_h_beta, l1_norm_e_gamma, l1_norm_e_beta):
    B, N, node_dim = nodes.shape
    H = _H
    ti = 32
    f32 = jnp.float32

    epk = jnp.swapaxes(edges, 2, 3)

    eye8 = jnp.eye(8, dtype=f32)
    wbig0 = jnp.kron(eye8, l0_C_w.T).astype(jnp.bfloat16)
    wbig1 = jnp.kron(eye8, l1_C_w.T).astype(jnp.bfloat16)
    wepk = jnp.concatenate(
        [init_edge_w, init_edge_b.reshape(1, H), l0_C_b.reshape(1, H)], axis=0)

    wu0 = jnp.concatenate([l0_U_w, l0_V_w, l0_A_w, l0_B_w], axis=1)
    bu0 = jnp.concatenate([l0_U_b, l0_V_b, l0_A_b, l0_B_b], axis=0)
    wu1 = jnp.concatenate([l1_U_w, l1_V_w, l1_A_w, l1_B_w], axis=1)
    bu1 = jnp.concatenate([l1_U_b, l1_V_b, l1_A_b, l1_B_b], axis=0)

    def t_slab(col):
        return jnp.swapaxes(col.reshape(B, N, H), 1, 2)

    h0_2d, uvab0 = _node_init(nodes.reshape(B * N, node_dim),
                              init_node_w, init_node_b, wu0, bu0)
    u0 = uvab0[:, 0:H].reshape(B, N, H)
    vht0 = t_slab(uvab0[:, H:2 * H])
    aht0 = t_slab(uvab0[:, 2 * H:3 * H]) + l0_C_b.reshape(1, H, 1)
    b0 = uvab0[:, 3 * H:4 * H].reshape(B, N, H)

    hpre0, st0 = _pass_a(epk, u0, b0, aht0, vht0, wbig0, wepk, ti=ti, N=N)
    hs0, ht0, es0, et0 = _bn_fold(st0, l0_norm_h_gamma, l0_norm_h_beta,
                                  l0_norm_e_gamma, l0_norm_e_beta, B, N)
    wbig0s = jnp.kron(eye8, (l0_C_w * es0[None, :]).T).astype(jnp.bfloat16)
    aht0s = aht0 * es0[None, :, None] + et0.reshape(1, H, 1)
    b0s = b0 * es0[None, None, :]

    h1_2d, uvab1 = _node_update(hpre0.reshape(B * N, H), h0_2d,
                                hs0, ht0, wu1, bu1)
    u1 = uvab1[:, 0:H].reshape(B, N, H)
    vht1 = t_slab(uvab1[:, H:2 * H])
    aht1 = t_slab(uvab1[:, 2 * H:3 * H]) + l1_C_b.reshape(1, H, 1)
    b1 = uvab1[:, 3 * H:4 * H].reshape(B, N, H)

    hpre1, st1 = _pass_b(epk, b0s, aht0s, wbig0s, wepk,
                         u1, b1, aht1, vht1, wbig1, ti=ti, N=N)
    hs1, ht1, es1, et1 = _bn_fold(st1, l1_norm_h_gamma, l1_norm_h_beta,
                                  l1_norm_e_gamma, l1_norm_e_beta, B, N)
    sh1 = jnp.stack([hs1, ht1], axis=0)
    wbig1s = jnp.kron(eye8, (l1_C_w * es1[None, :]).T).astype(jnp.bfloat16)
    aht1s = aht1 * es1[None, :, None] + et1.reshape(1, H, 1)
    b1s = b1 * es1[None, None, :]

    e_out_t, h_out = _pass_c(epk, b0s, aht0s, wbig0s, wepk,
                             b1s, aht1s, wbig1s,
                             hpre1, h1_2d.reshape(B, N, H), sh1,
                             ti=ti, N=N)
    return h_out, jnp.swapaxes(e_out_t, 2, 3)


# speedup vs baseline: 6.5693x; 1.1844x over previous
import functools

import jax
import jax.numpy as jnp
from jax.experimental import pallas as pl
from jax.experimental.pallas import tpu as pltpu

_VMEM_LIMIT = 48 * 1024 * 1024
_H = 32


def _node_init_body(x_ref, wn_ref, bn_ref, wu_ref, bu_ref, h_ref, uvab_ref):
    h = (jnp.dot(x_ref[...], wn_ref[...], preferred_element_type=jnp.float32)
         + bn_ref[...])
    h_ref[...] = h
    uvab_ref[...] = (
        jnp.dot(h, wu_ref[...], preferred_element_type=jnp.float32)
        + bu_ref[...])


def _node_init(x2d, wn, bn, wu, bu):
    m, k = x2d.shape
    tm = m // 8 if m % 8 == 0 else m
    return pl.pallas_call(
        _node_init_body,
        grid=(m // tm,),
        out_shape=(jax.ShapeDtypeStruct((m, _H), jnp.float32),
                   jax.ShapeDtypeStruct((m, 4 * _H), jnp.float32)),
        in_specs=[
            pl.BlockSpec((tm, k), lambda i: (i, 0)),
            pl.BlockSpec((k, _H), lambda i: (0, 0)),
            pl.BlockSpec((1, _H), lambda i: (0, 0)),
            pl.BlockSpec((_H, 4 * _H), lambda i: (0, 0)),
            pl.BlockSpec((1, 4 * _H), lambda i: (0, 0)),
        ],
        out_specs=(pl.BlockSpec((tm, _H), lambda i: (i, 0)),
                   pl.BlockSpec((tm, 4 * _H), lambda i: (i, 0))),
        compiler_params=pltpu.CompilerParams(
            dimension_semantics=("parallel",), vmem_limit_bytes=_VMEM_LIMIT),
    )(x2d, wn, bn.reshape(1, _H), wu, bu.reshape(1, 4 * _H))


def _node_update_body(hp_ref, hr_ref, s_ref, t_ref, wu_ref, bu_ref,
                      h_ref, uvab_ref):
    y = hp_ref[...] * s_ref[...] + t_ref[...]
    h = hr_ref[...] + jnp.maximum(y, 0.0)
    h_ref[...] = h
    uvab_ref[...] = (
        jnp.dot(h, wu_ref[...], preferred_element_type=jnp.float32)
        + bu_ref[...])


def _node_update(hpre2d, hres2d, scale, shift, wu, bu):
    m = hpre2d.shape[0]
    tm = m // 8 if m % 8 == 0 else m
    row = pl.BlockSpec((tm, _H), lambda i: (i, 0))
    vec = pl.BlockSpec((1, _H), lambda i: (0, 0))
    return pl.pallas_call(
        _node_update_body,
        grid=(m // tm,),
        out_shape=(jax.ShapeDtypeStruct((m, _H), jnp.float32),
                   jax.ShapeDtypeStruct((m, 4 * _H), jnp.float32)),
        in_specs=[row, row, vec, vec,
                  pl.BlockSpec((_H, 4 * _H), lambda i: (0, 0)),
                  pl.BlockSpec((1, 4 * _H), lambda i: (0, 0))],
        out_specs=(row, pl.BlockSpec((tm, 4 * _H), lambda i: (i, 0))),
        compiler_params=pltpu.CompilerParams(
            dimension_semantics=("parallel",), vmem_limit_bytes=_VMEM_LIMIT),
    )(hpre2d, hres2d, scale.reshape(1, _H), shift.reshape(1, _H),
      wu, bu.reshape(1, 4 * _H))


def _embed_edges_t(epk_ref, we, be, ti, N):
    x0 = epk_ref[0, :, 0, :]
    x1 = epk_ref[0, :, 1, :]
    w0 = we[0].reshape(1, _H, 1)
    w1 = we[1].reshape(1, _H, 1)
    return (x0[:, None, :] * w0 + x1[:, None, :] * w1
            + be.reshape(1, _H, 1))


def _cproj_t(e_t3, wbig_ref, ti, N):
    e2 = e_t3.reshape(ti * _H, N).astype(jnp.bfloat16)
    w = wbig_ref[...]
    parts = [
        jnp.dot(w, e2[s * 8 * _H:(s + 1) * 8 * _H, :],
                preferred_element_type=jnp.float32)
        for s in range(ti // 8)
    ]
    ce = jnp.concatenate(parts, axis=0) if len(parts) > 1 else parts[0]
    return ce.reshape(ti, _H, N)


def _gate_aggregate_t(e_new, vh_t, u, N):
    gated = jax.nn.sigmoid(e_new) * vh_t[None, :, :]
    return u + jnp.sum(gated, axis=2) * (1.0 / N)


def _write_stats_t(st_ref, h_pre, e_new):
    es = jnp.sum(e_new, axis=(0, 2)).reshape(1, _H)
    eq = jnp.sum(e_new * e_new, axis=(0, 2)).reshape(1, _H)
    st_ref[0, 0] = jnp.concatenate(
        [jnp.sum(h_pre, axis=0, keepdims=True),
         jnp.sum(h_pre * h_pre, axis=0, keepdims=True), es, eq], axis=0)


def _pass_a_body(epk_ref, u_ref, b_ref, aht_ref, vht_ref, wbig_ref,
                 we_ref, hpre_ref, st_ref, *, ti, N):
    e0 = _embed_edges_t(epk_ref, we_ref[0:2, :], we_ref[2, :], ti, N)
    ce = _cproj_t(e0, wbig_ref, ti, N)
    e_new = ce + aht_ref[0][None, :, :] + b_ref[0][:, :, None]
    h_pre = _gate_aggregate_t(e_new, vht_ref[0], u_ref[0], N)
    hpre_ref[0] = h_pre
    _write_stats_t(st_ref, h_pre, e_new)


def _pass_a(epk, u, b, aht, vht, wbig, wepk, *, ti, N):
    B = epk.shape[0]
    n_it = N // ti
    body = functools.partial(_pass_a_body, ti=ti, N=N)
    rows = pl.BlockSpec((1, ti, _H), lambda bb, it: (bb, it, 0))
    slab = pl.BlockSpec((1, _H, N), lambda bb, it: (bb, 0, 0))
    return pl.pallas_call(
        body,
        grid=(B, n_it),
        out_shape=(jax.ShapeDtypeStruct((B, N, _H), jnp.float32),
                   jax.ShapeDtypeStruct((B, n_it, 4, _H), jnp.float32)),
        in_specs=[
            pl.BlockSpec((1, ti, 2, N), lambda bb, it: (bb, it, 0, 0)),
            rows, rows, slab, slab,
            pl.BlockSpec((8 * _H, 8 * _H), lambda bb, it: (0, 0)),
            pl.BlockSpec((4, _H), lambda bb, it: (0, 0)),
        ],
        out_specs=(
            rows,
            pl.BlockSpec((1, 1, 4, _H), lambda bb, it: (bb, it, 0, 0)),
        ),
        compiler_params=pltpu.CompilerParams(
            dimension_semantics=("parallel", "parallel"),
            vmem_limit_bytes=_VMEM_LIMIT),
    )(epk, u, b, aht, vht, wbig, wepk)


def _pass_b_body(epk_ref, b0s_ref, aht0s_ref, wbig0s_ref, we_ref,
                 u1_ref, b1_ref, aht1_ref, vht1_ref, wbig1_ref,
                 hpre_ref, st_ref, *, ti, N):
    e0 = _embed_edges_t(epk_ref, we_ref[0:2, :], we_ref[2, :], ti, N)
    ce0s = _cproj_t(e0, wbig0s_ref, ti, N)
    e_new0s = ce0s + aht0s_ref[0][None, :, :] + b0s_ref[0][:, :, None]
    e1 = e0 + jnp.maximum(e_new0s, 0.0)
    ce1 = _cproj_t(e1, wbig1_ref, ti, N)
    e_new1 = ce1 + aht1_ref[0][None, :, :] + b1_ref[0][:, :, None]
    h_pre = _gate_aggregate_t(e_new1, vht1_ref[0], u1_ref[0], N)
    hpre_ref[0] = h_pre
    _write_stats_t(st_ref, h_pre, e_new1)


def _pass_b(epk, b0s, aht0s, wbig0s, wepk, u1, b1, aht1, vht1, wbig1,
            *, ti, N):
    B = epk.shape[0]
    n_it = N // ti
    body = functools.partial(_pass_b_body, ti=ti, N=N)
    rows = pl.BlockSpec((1, ti, _H), lambda bb, it: (bb, it, 0))
    slab = pl.BlockSpec((1, _H, N), lambda bb, it: (bb, 0, 0))
    wfull = pl.BlockSpec((8 * _H, 8 * _H), lambda bb, it: (0, 0))
    return pl.pallas_call(
        body,
        grid=(B, n_it),
        out_shape=(jax.ShapeDtypeStruct((B, N, _H), jnp.float32),
                   jax.ShapeDtypeStruct((B, n_it, 4, _H), jnp.float32)),
        in_specs=[
            pl.BlockSpec((1, ti, 2, N), lambda bb, it: (bb, it, 0, 0)),
            rows, slab, wfull,
            pl.BlockSpec((4, _H), lambda bb, it: (0, 0)),
            rows, rows, slab, slab, wfull,
        ],
        out_specs=(
            rows,
            pl.BlockSpec((1, 1, 4, _H), lambda bb, it: (bb, it, 0, 0)),
        ),
        compiler_params=pltpu.CompilerParams(
            dimension_semantics=("parallel", "parallel"),
            vmem_limit_bytes=_VMEM_LIMIT),
    )(epk, b0s, aht0s, wbig0s, wepk, u1, b1, aht1, vht1, wbig1)


def _pass_c_body(epk_ref, b0s_ref, aht0s_ref, wbig0s_ref, we_ref,
                 b1s_ref, aht1s_ref, wbig1s_ref,
                 hp1_ref, hr_ref, sh1_ref,
                 eout_ref, hout_ref, *, ti, N):
    e0 = _embed_edges_t(epk_ref, we_ref[0:2, :], we_ref[2, :], ti, N)
    ce0s = _cproj_t(e0, wbig0s_ref, ti, N)
    e_new0s = ce0s + aht0s_ref[0][None, :, :] + b0s_ref[0][:, :, None]
    e1 = e0 + jnp.maximum(e_new0s, 0.0)
    ce1s = _cproj_t(e1, wbig1s_ref, ti, N)
    e_new1s = ce1s + aht1s_ref[0][None, :, :] + b1s_ref[0][:, :, None]
    eout_ref[0] = e1 + jnp.maximum(e_new1s, 0.0)
    yh = hp1_ref[0] * sh1_ref[0:1, :] + sh1_ref[1:2, :]
    hout_ref[0] = hr_ref[0] + jnp.maximum(yh, 0.0)


def _pass_c(epk, b0s, aht0s, wbig0s, wepk, b1s, aht1s, wbig1s,
            hpre1, hres, sh1, *, ti, N):
    B = epk.shape[0]
    n_it = N // ti
    body = functools.partial(_pass_c_body, ti=ti, N=N)
    rows = pl.BlockSpec((1, ti, _H), lambda bb, it: (bb, it, 0))
    slab = pl.BlockSpec((1, _H, N), lambda bb, it: (bb, 0, 0))
    wfull = pl.BlockSpec((8 * _H, 8 * _H), lambda bb, it: (0, 0))
    return pl.pallas_call(
        body,
        grid=(B, n_it),
        out_shape=(jax.ShapeDtypeStruct((B, N, _H, N), jnp.float32),
                   jax.ShapeDtypeStruct((B, N, _H), jnp.float32)),
        in_specs=[
            pl.BlockSpec((1, ti, 2, N), lambda bb, it: (bb, it, 0, 0)),
            rows, slab, wfull,
            pl.BlockSpec((4, _H), lambda bb, it: (0, 0)),
            rows, slab, wfull,
            rows, rows,
            pl.BlockSpec((2, _H), lambda bb, it: (0, 0)),
        ],
        out_specs=(
            pl.BlockSpec((1, ti, _H, N), lambda bb, it: (bb, it, 0, 0)),
            rows,
        ),
        compiler_params=pltpu.CompilerParams(
            dimension_semantics=("parallel", "parallel"),
            vmem_limit_bytes=_VMEM_LIMIT),
    )(epk, b0s, aht0s, wbig0s, wepk, b1s, aht1s, wbig1s,
      hpre1, hres, sh1)


def _bn_fold(stats, gamma_h, beta_h, gamma_e, beta_e, B, N, eps=1e-5):
    mh = float(B * N)
    h_mean = jnp.sum(stats[:, :, 0, :], axis=(0, 1)) / mh
    h_var = jnp.maximum(
        jnp.sum(stats[:, :, 1, :], axis=(0, 1)) / mh - h_mean * h_mean, 0.0)
    h_scale = gamma_h * jax.lax.rsqrt(h_var + eps)
    h_shift = beta_h - h_mean * h_scale
    me = float(B * N * N)
    e_mean = jnp.sum(stats[:, :, 2, :], axis=(0, 1)) / me
    e_var = jnp.maximum(
        jnp.sum(stats[:, :, 3, :], axis=(0, 1)) / me - e_mean * e_mean, 0.0)
    e_scale = gamma_e * jax.lax.rsqrt(e_var + eps)
    e_shift = beta_e - e_mean * e_scale
    return h_scale, h_shift, e_scale, e_shift


def kernel(nodes, edges,
           init_node_w, init_node_b, init_edge_w, init_edge_b,
           l0_U_w, l0_U_b, l0_V_w, l0_V_b, l0_A_w, l0_A_b,
           l0_B_w, l0_B_b, l0_C_w, l0_C_b,
           l0_norm_h_gamma, l0_norm_h_beta, l0_norm_e_gamma, l0_norm_e_beta,
           l1_U_w, l1_U_b, l1_V_w, l1_V_b, l1_A_w, l1_A_b,
           l1_B_w, l1_B_b, l1_C_w, l1_C_b,
           l1_norm_h_gamma, l1_norm_h_beta, l1_norm_e_gamma, l1_norm_e_beta):
    B, N, node_dim = nodes.shape
    H = _H
    ti = 64
    f32 = jnp.float32

    epk = jnp.swapaxes(edges, 2, 3)

    eye8 = jnp.eye(8, dtype=f32)
    wbig0 = jnp.kron(eye8, l0_C_w.T).astype(jnp.bfloat16)
    wbig1 = jnp.kron(eye8, l1_C_w.T).astype(jnp.bfloat16)
    wepk = jnp.concatenate(
        [init_edge_w, init_edge_b.reshape(1, H), l0_C_b.reshape(1, H)], axis=0)

    wu0 = jnp.concatenate([l0_U_w, l0_V_w, l0_A_w, l0_B_w], axis=1)
    bu0 = jnp.concatenate([l0_U_b, l0_V_b, l0_A_b, l0_B_b], axis=0)
    wu1 = jnp.concatenate([l1_U_w, l1_V_w, l1_A_w, l1_B_w], axis=1)
    bu1 = jnp.concatenate([l1_U_b, l1_V_b, l1_A_b, l1_B_b], axis=0)

    def t_slab(col):
        return jnp.swapaxes(col.reshape(B, N, H), 1, 2)

    h0_2d, uvab0 = _node_init(nodes.reshape(B * N, node_dim),
                              init_node_w, init_node_b, wu0, bu0)
    u0 = uvab0[:, 0:H].reshape(B, N, H)
    vht0 = t_slab(uvab0[:, H:2 * H])
    aht0 = t_slab(uvab0[:, 2 * H:3 * H]) + l0_C_b.reshape(1, H, 1)
    b0 = uvab0[:, 3 * H:4 * H].reshape(B, N, H)

    hpre0, st0 = _pass_a(epk, u0, b0, aht0, vht0, wbig0, wepk, ti=ti, N=N)
    hs0, ht0, es0, et0 = _bn_fold(st0, l0_norm_h_gamma, l0_norm_h_beta,
                                  l0_norm_e_gamma, l0_norm_e_beta, B, N)
    wbig0s = jnp.kron(eye8, (l0_C_w * es0[None, :]).T).astype(jnp.bfloat16)
    aht0s = aht0 * es0[None, :, None] + et0.reshape(1, H, 1)
    b0s = b0 * es0[None, None, :]

    h1_2d, uvab1 = _node_update(hpre0.reshape(B * N, H), h0_2d,
                                hs0, ht0, wu1, bu1)
    u1 = uvab1[:, 0:H].reshape(B, N, H)
    vht1 = t_slab(uvab1[:, H:2 * H])
    aht1 = t_slab(uvab1[:, 2 * H:3 * H]) + l1_C_b.reshape(1, H, 1)
    b1 = uvab1[:, 3 * H:4 * H].reshape(B, N, H)

    hpre1, st1 = _pass_b(epk, b0s, aht0s, wbig0s, wepk,
                         u1, b1, aht1, vht1, wbig1, ti=ti, N=N)
    hs1, ht1, es1, et1 = _bn_fold(st1, l1_norm_h_gamma, l1_norm_h_beta,
                                  l1_norm_e_gamma, l1_norm_e_beta, B, N)
    sh1 = jnp.stack([hs1, ht1], axis=0)
    wbig1s = jnp.kron(eye8, (l1_C_w * es1[None, :]).T).astype(jnp.bfloat16)
    aht1s = aht1 * es1[None, :, None] + et1.reshape(1, H, 1)
    b1s = b1 * es1[None, None, :]

    e_out_t, h_out = _pass_c(epk, b0s, aht0s, wbig0s, wepk,
                             b1s, aht1s, wbig1s,
                             hpre1, h1_2d.reshape(B, N, H), sh1,
                             ti=ti, N=N)
    return h_out, jnp.swapaxes(e_out_t, 2, 3)


# speedup vs baseline: 6.6137x; 1.0067x over previous
import functools

import jax
import jax.numpy as jnp
from jax.experimental import pallas as pl
from jax.experimental.pallas import tpu as pltpu

_VMEM_LIMIT = 48 * 1024 * 1024
_H = 32


def _node_init_body(x_ref, wn_ref, bn_ref, wu_ref, bu_ref, h_ref, uvab_ref):
    h = (jnp.dot(x_ref[...], wn_ref[...], preferred_element_type=jnp.float32)
         + bn_ref[...])
    h_ref[...] = h
    uvab_ref[...] = (
        jnp.dot(h, wu_ref[...], preferred_element_type=jnp.float32)
        + bu_ref[...])


def _node_init(x2d, wn, bn, wu, bu):
    m, k = x2d.shape
    tm = m // 8 if m % 8 == 0 else m
    return pl.pallas_call(
        _node_init_body,
        grid=(m // tm,),
        out_shape=(jax.ShapeDtypeStruct((m, _H), jnp.float32),
                   jax.ShapeDtypeStruct((m, 4 * _H), jnp.float32)),
        in_specs=[
            pl.BlockSpec((tm, k), lambda i: (i, 0)),
            pl.BlockSpec((k, _H), lambda i: (0, 0)),
            pl.BlockSpec((1, _H), lambda i: (0, 0)),
            pl.BlockSpec((_H, 4 * _H), lambda i: (0, 0)),
            pl.BlockSpec((1, 4 * _H), lambda i: (0, 0)),
        ],
        out_specs=(pl.BlockSpec((tm, _H), lambda i: (i, 0)),
                   pl.BlockSpec((tm, 4 * _H), lambda i: (i, 0))),
        compiler_params=pltpu.CompilerParams(
            dimension_semantics=("parallel",), vmem_limit_bytes=_VMEM_LIMIT),
    )(x2d, wn, bn.reshape(1, _H), wu, bu.reshape(1, 4 * _H))


def _node_update_body(hp_ref, hr_ref, s_ref, t_ref, wu_ref, bu_ref,
                      h_ref, uvab_ref):
    y = hp_ref[...] * s_ref[...] + t_ref[...]
    h = hr_ref[...] + jnp.maximum(y, 0.0)
    h_ref[...] = h
    uvab_ref[...] = (
        jnp.dot(h, wu_ref[...], preferred_element_type=jnp.float32)
        + bu_ref[...])


def _node_update(hpre2d, hres2d, scale, shift, wu, bu):
    m = hpre2d.shape[0]
    tm = m // 8 if m % 8 == 0 else m
    row = pl.BlockSpec((tm, _H), lambda i: (i, 0))
    vec = pl.BlockSpec((1, _H), lambda i: (0, 0))
    return pl.pallas_call(
        _node_update_body,
        grid=(m // tm,),
        out_shape=(jax.ShapeDtypeStruct((m, _H), jnp.float32),
                   jax.ShapeDtypeStruct((m, 4 * _H), jnp.float32)),
        in_specs=[row, row, vec, vec,
                  pl.BlockSpec((_H, 4 * _H), lambda i: (0, 0)),
                  pl.BlockSpec((1, 4 * _H), lambda i: (0, 0))],
        out_specs=(row, pl.BlockSpec((tm, 4 * _H), lambda i: (i, 0))),
        compiler_params=pltpu.CompilerParams(
            dimension_semantics=("parallel",), vmem_limit_bytes=_VMEM_LIMIT),
    )(hpre2d, hres2d, scale.reshape(1, _H), shift.reshape(1, _H),
      wu, bu.reshape(1, 4 * _H))


def _embed_edges_t(epk_ref, we, be, ti, N):
    x0 = epk_ref[0, :, 0, :]
    x1 = epk_ref[0, :, 1, :]
    w0 = we[0].reshape(1, _H, 1)
    w1 = we[1].reshape(1, _H, 1)
    return (x0[:, None, :] * w0 + x1[:, None, :] * w1
            + be.reshape(1, _H, 1))


def _cproj_t(e_t3, wbig_ref, ti, N):
    e2 = e_t3.reshape(ti * _H, N).astype(jnp.bfloat16)
    w = wbig_ref[...]
    parts = [
        jnp.dot(w, e2[s * 8 * _H:(s + 1) * 8 * _H, :],
                preferred_element_type=jnp.float32)
        for s in range(ti // 8)
    ]
    ce = jnp.concatenate(parts, axis=0) if len(parts) > 1 else parts[0]
    return ce.reshape(ti, _H, N)


def _gate_aggregate_t(e_new, vh_t, u, N):
    gated = jax.nn.sigmoid(e_new) * vh_t[None, :, :]
    return u + jnp.sum(gated, axis=2) * (1.0 / N)


def _write_stats_t(st_ref, h_pre, e_new):
    es = jnp.sum(e_new, axis=(0, 2)).reshape(1, _H)
    eq = jnp.sum(e_new * e_new, axis=(0, 2)).reshape(1, _H)
    st_ref[0, 0] = jnp.concatenate(
        [jnp.sum(h_pre, axis=0, keepdims=True),
         jnp.sum(h_pre * h_pre, axis=0, keepdims=True), es, eq], axis=0)


def _pass_a_body(epk_ref, u_ref, b_ref, aht_ref, vht_ref, wbig_ref,
                 we_ref, hpre_ref, st_ref, *, ti, N):
    e0 = _embed_edges_t(epk_ref, we_ref[0:2, :], we_ref[2, :], ti, N)
    ce = _cproj_t(e0, wbig_ref, ti, N)
    e_new = ce + aht_ref[0][None, :, :] + b_ref[0][:, :, None]
    h_pre = _gate_aggregate_t(e_new, vht_ref[0], u_ref[0], N)
    hpre_ref[0] = h_pre
    _write_stats_t(st_ref, h_pre, e_new)


def _pass_a(epk, u, b, aht, vht, wbig, wepk, *, ti, N):
    B = epk.shape[0]
    n_it = N // ti
    body = functools.partial(_pass_a_body, ti=ti, N=N)
    rows = pl.BlockSpec((1, ti, _H), lambda bb, it: (bb, it, 0))
    slab = pl.BlockSpec((1, _H, N), lambda bb, it: (bb, 0, 0))
    return pl.pallas_call(
        body,
        grid=(B, n_it),
        out_shape=(jax.ShapeDtypeStruct((B, N, _H), jnp.float32),
                   jax.ShapeDtypeStruct((B, n_it, 4, _H), jnp.float32)),
        in_specs=[
            pl.BlockSpec((1, ti, 2, N), lambda bb, it: (bb, it, 0, 0)),
            rows, rows, slab, slab,
            pl.BlockSpec((8 * _H, 8 * _H), lambda bb, it: (0, 0)),
            pl.BlockSpec((4, _H), lambda bb, it: (0, 0)),
        ],
        out_specs=(
            rows,
            pl.BlockSpec((1, 1, 4, _H), lambda bb, it: (bb, it, 0, 0)),
        ),
        compiler_params=pltpu.CompilerParams(
            dimension_semantics=("parallel", "parallel"),
            vmem_limit_bytes=_VMEM_LIMIT),
    )(epk, u, b, aht, vht, wbig, wepk)


def _pass_b_body(epk_ref, b0s_ref, aht0s_ref, wbig0s_ref, we_ref,
                 u1_ref, b1_ref, aht1_ref, vht1_ref, wbig1_ref,
                 hpre_ref, st_ref, *, ti, N):
    e0 = _embed_edges_t(epk_ref, we_ref[0:2, :], we_ref[2, :], ti, N)
    ce0s = _cproj_t(e0, wbig0s_ref, ti, N)
    e_new0s = ce0s + aht0s_ref[0][None, :, :] + b0s_ref[0][:, :, None]
    e1 = e0 + jnp.maximum(e_new0s, 0.0)
    ce1 = _cproj_t(e1, wbig1_ref, ti, N)
    e_new1 = ce1 + aht1_ref[0][None, :, :] + b1_ref[0][:, :, None]
    h_pre = _gate_aggregate_t(e_new1, vht1_ref[0], u1_ref[0], N)
    hpre_ref[0] = h_pre
    _write_stats_t(st_ref, h_pre, e_new1)


def _pass_b(epk, b0s, aht0s, wbig0s, wepk, u1, b1, aht1, vht1, wbig1,
            *, ti, N):
    B = epk.shape[0]
    n_it = N // ti
    body = functools.partial(_pass_b_body, ti=ti, N=N)
    rows = pl.BlockSpec((1, ti, _H), lambda bb, it: (bb, it, 0))
    slab = pl.BlockSpec((1, _H, N), lambda bb, it: (bb, 0, 0))
    wfull = pl.BlockSpec((8 * _H, 8 * _H), lambda bb, it: (0, 0))
    return pl.pallas_call(
        body,
        grid=(B, n_it),
        out_shape=(jax.ShapeDtypeStruct((B, N, _H), jnp.float32),
                   jax.ShapeDtypeStruct((B, n_it, 4, _H), jnp.float32)),
        in_specs=[
            pl.BlockSpec((1, ti, 2, N), lambda bb, it: (bb, it, 0, 0)),
            rows, slab, wfull,
            pl.BlockSpec((4, _H), lambda bb, it: (0, 0)),
            rows, rows, slab, slab, wfull,
        ],
        out_specs=(
            rows,
            pl.BlockSpec((1, 1, 4, _H), lambda bb, it: (bb, it, 0, 0)),
        ),
        compiler_params=pltpu.CompilerParams(
            dimension_semantics=("parallel", "parallel"),
            vmem_limit_bytes=_VMEM_LIMIT),
    )(epk, b0s, aht0s, wbig0s, wepk, u1, b1, aht1, vht1, wbig1)


def _pass_c_body(epk_ref, b0s_ref, aht0s_ref, wbig0s_ref, we_ref,
                 b1s_ref, aht1s_ref, wbig1s_ref,
                 hp1_ref, hr_ref, sh1_ref,
                 eout_ref, hout_ref, *, ti, N):
    e0 = _embed_edges_t(epk_ref, we_ref[0:2, :], we_ref[2, :], ti, N)
    ce0s = _cproj_t(e0, wbig0s_ref, ti, N)
    e_new0s = ce0s + aht0s_ref[0][None, :, :] + b0s_ref[0][:, :, None]
    e1 = e0 + jnp.maximum(e_new0s, 0.0)
    ce1s = _cproj_t(e1, wbig1s_ref, ti, N)
    e_new1s = ce1s + aht1s_ref[0][None, :, :] + b1s_ref[0][:, :, None]
    eout_ref[0] = e1 + jnp.maximum(e_new1s, 0.0)
    yh = hp1_ref[0] * sh1_ref[0:1, :] + sh1_ref[1:2, :]
    hout_ref[0] = hr_ref[0] + jnp.maximum(yh, 0.0)


def _pass_c(epk, b0s, aht0s, wbig0s, wepk, b1s, aht1s, wbig1s,
            hpre1, hres, sh1, *, ti, N):
    B = epk.shape[0]
    n_it = N // ti
    body = functools.partial(_pass_c_body, ti=ti, N=N)
    rows = pl.BlockSpec((1, ti, _H), lambda bb, it: (bb, it, 0))
    slab = pl.BlockSpec((1, _H, N), lambda bb, it: (bb, 0, 0))
    wfull = pl.BlockSpec((8 * _H, 8 * _H), lambda bb, it: (0, 0))
    return pl.pallas_call(
        body,
        grid=(B, n_it),
        out_shape=(jax.ShapeDtypeStruct((B, N, _H, N), jnp.float32),
                   jax.ShapeDtypeStruct((B, N, _H), jnp.float32)),
        in_specs=[
            pl.BlockSpec((1, ti, 2, N), lambda bb, it: (bb, it, 0, 0)),
            rows, slab, wfull,
            pl.BlockSpec((4, _H), lambda bb, it: (0, 0)),
            rows, slab, wfull,
            rows, rows,
            pl.BlockSpec((2, _H), lambda bb, it: (0, 0)),
        ],
        out_specs=(
            pl.BlockSpec((1, ti, _H, N), lambda bb, it: (bb, it, 0, 0)),
            rows,
        ),
        compiler_params=pltpu.CompilerParams(
            dimension_semantics=("parallel", "parallel"),
            vmem_limit_bytes=_VMEM_LIMIT),
    )(epk, b0s, aht0s, wbig0s, wepk, b1s, aht1s, wbig1s,
      hpre1, hres, sh1)


def _bn_fold(stats, gamma_h, beta_h, gamma_e, beta_e, B, N, eps=1e-5):
    mh = float(B * N)
    h_mean = jnp.sum(stats[:, :, 0, :], axis=(0, 1)) / mh
    h_var = jnp.maximum(
        jnp.sum(stats[:, :, 1, :], axis=(0, 1)) / mh - h_mean * h_mean, 0.0)
    h_scale = gamma_h * jax.lax.rsqrt(h_var + eps)
    h_shift = beta_h - h_mean * h_scale
    me = float(B * N * N)
    e_mean = jnp.sum(stats[:, :, 2, :], axis=(0, 1)) / me
    e_var = jnp.maximum(
        jnp.sum(stats[:, :, 3, :], axis=(0, 1)) / me - e_mean * e_mean, 0.0)
    e_scale = gamma_e * jax.lax.rsqrt(e_var + eps)
    e_shift = beta_e - e_mean * e_scale
    return h_scale, h_shift, e_scale, e_shift


def kernel(nodes, edges,
           init_node_w, init_node_b, init_edge_w, init_edge_b,
           l0_U_w, l0_U_b, l0_V_w, l0_V_b, l0_A_w, l0_A_b,
           l0_B_w, l0_B_b, l0_C_w, l0_C_b,
           l0_norm_h_gamma, l0_norm_h_beta, l0_norm_e_gamma, l0_norm_e_beta,
           l1_U_w, l1_U_b, l1_V_w, l1_V_b, l1_A_w, l1_A_b,
           l1_B_w, l1_B_b, l1_C_w, l1_C_b,
           l1_norm_h_gamma, l1_norm_h_beta, l1_norm_e_gamma, l1_norm_e_beta):
    B, N, node_dim = nodes.shape
    H = _H
    ti = 128
    f32 = jnp.float32

    epk = jnp.swapaxes(edges, 2, 3)

    eye8 = jnp.eye(8, dtype=f32)
    wbig0 = jnp.kron(eye8, l0_C_w.T).astype(jnp.bfloat16)
    wbig1 = jnp.kron(eye8, l1_C_w.T).astype(jnp.bfloat16)
    wepk = jnp.concatenate(
        [init_edge_w, init_edge_b.reshape(1, H), l0_C_b.reshape(1, H)], axis=0)

    wu0 = jnp.concatenate([l0_U_w, l0_V_w, l0_A_w, l0_B_w], axis=1)
    bu0 = jnp.concatenate([l0_U_b, l0_V_b, l0_A_b, l0_B_b], axis=0)
    wu1 = jnp.concatenate([l1_U_w, l1_V_w, l1_A_w, l1_B_w], axis=1)
    bu1 = jnp.concatenate([l1_U_b, l1_V_b, l1_A_b, l1_B_b], axis=0)

    def t_slab(col):
        return jnp.swapaxes(col.reshape(B, N, H), 1, 2)

    h0_2d, uvab0 = _node_init(nodes.reshape(B * N, node_dim),
                              init_node_w, init_node_b, wu0, bu0)
    u0 = uvab0[:, 0:H].reshape(B, N, H)
    vht0 = t_slab(uvab0[:, H:2 * H])
    aht0 = t_slab(uvab0[:, 2 * H:3 * H]) + l0_C_b.reshape(1, H, 1)
    b0 = uvab0[:, 3 * H:4 * H].reshape(B, N, H)

    hpre0, st0 = _pass_a(epk, u0, b0, aht0, vht0, wbig0, wepk, ti=ti, N=N)
    hs0, ht0, es0, et0 = _bn_fold(st0, l0_norm_h_gamma, l0_norm_h_beta,
                                  l0_norm_e_gamma, l0_norm_e_beta, B, N)
    wbig0s = jnp.kron(eye8, (l0_C_w * es0[None, :]).T).astype(jnp.bfloat16)
    aht0s = aht0 * es0[None, :, None] + et0.reshape(1, H, 1)
    b0s = b0 * es0[None, None, :]

    h1_2d, uvab1 = _node_update(hpre0.reshape(B * N, H), h0_2d,
                                hs0, ht0, wu1, bu1)
    u1 = uvab1[:, 0:H].reshape(B, N, H)
    vht1 = t_slab(uvab1[:, H:2 * H])
    aht1 = t_slab(uvab1[:, 2 * H:3 * H]) + l1_C_b.reshape(1, H, 1)
    b1 = uvab1[:, 3 * H:4 * H].reshape(B, N, H)

    hpre1, st1 = _pass_b(epk, b0s, aht0s, wbig0s, wepk,
                         u1, b1, aht1, vht1, wbig1, ti=ti, N=N)
    hs1, ht1, es1, et1 = _bn_fold(st1, l1_norm_h_gamma, l1_norm_h_beta,
                                  l1_norm_e_gamma, l1_norm_e_beta, B, N)
    sh1 = jnp.stack([hs1, ht1], axis=0)
    wbig1s = jnp.kron(eye8, (l1_C_w * es1[None, :]).T).astype(jnp.bfloat16)
    aht1s = aht1 * es1[None, :, None] + et1.reshape(1, H, 1)
    b1s = b1 * es1[None, None, :]

    e_out_t, h_out = _pass_c(epk, b0s, aht0s, wbig0s, wepk,
                             b1s, aht1s, wbig1s,
                             hpre1, h1_2d.reshape(B, N, H), sh1,
                             ti=ti, N=N)
    return h_out, jnp.swapaxes(e_out_t, 2, 3)
